# baseline (speedup 1.0000x reference)
import functools
import jax
import jax.numpy as jnp
from jax import lax
from jax.experimental import pallas as pl
from jax.experimental.pallas import tpu as pltpu
from jax.experimental.pallas import tpu_sc as plsc

N_NODES = 10000
D_NODE = 128
D_EDGE = 16
H = 128
NUM_GRAPHS = 64

NC = 2
NS = 16
NW = NC * NS
CH = 64
LANES = 16

NROWS = 10112
DUMMY = N_NODES

_SC_PARAMS = pltpu.CompilerParams(
    needs_layout_passes=False, use_tc_tiling_on_sc=False)


def _sc_mesh():
  return plsc.VectorSubcoreMesh(core_axis_name="c", subcore_axis_name="s",
                                num_cores=NC, num_subcores=NS)


def _pad_edges(e_src, e_dst, n_pad):
  pe = n_pad - e_src.shape[0]
  src = jnp.concatenate([e_src, jnp.zeros((pe,), jnp.int32)])
  dst = jnp.concatenate([e_dst, jnp.full((pe,), DUMMY, jnp.int32)])
  return src, dst


def _epad(e):
  per = NW * CH
  return ((e + per - 1) // per) * per


def _embed_body(x_ref, wn_ref, bn_ref, out_ref):
  z = jnp.dot(x_ref[...], wn_ref[...], preferred_element_type=jnp.float32)
  z = z + bn_ref[...]
  out_ref[...] = z / (1.0 + jnp.exp(-z))


def _tc_embed(x, wn, bn):
  return pl.pallas_call(
      _embed_body,
      out_shape=jax.ShapeDtypeStruct((N_NODES, H), jnp.float32),
  )(x, wn, bn.reshape(1, H))


def _edgefeat_body(ea_ref, we_ref, bi_ref, o0, o1, o2, o3):
  ea = ea_ref[...]
  outs = (o0, o1, o2, o3)
  for l in range(4):
    z = jnp.dot(ea, we_ref[l], preferred_element_type=jnp.float32)
    outs[l][...] = z + bi_ref[l]


def _tc_edgefeat(ea_pad, we_all, bi_all, e_pad):
  blk = 4096
  grid = e_pad // blk
  outs = [jax.ShapeDtypeStruct((e_pad, H), jnp.float32)] * 4
  return pl.pallas_call(
      _edgefeat_body,
      grid=(grid,),
      in_specs=[
          pl.BlockSpec((blk, D_EDGE), lambda i: (i, 0)),
          pl.BlockSpec((4, D_EDGE, H), lambda i: (0, 0, 0)),
          pl.BlockSpec((4, 1, H), lambda i: (0, 0, 0)),
      ],
      out_specs=[pl.BlockSpec((blk, H), lambda i: (i, 0))] * 4,
      out_shape=outs,
  )(ea_pad, we_all, bi_all)


def _proj_body(h_ref, wi_ref, ws_ref, bs_ref, oi_ref, os_ref):
  h = h_ref[...]
  oi_ref[...] = jnp.dot(h, wi_ref[...], preferred_element_type=jnp.float32)
  os_ref[...] = (jnp.dot(h, ws_ref[...], preferred_element_type=jnp.float32)
                 + bs_ref[...])


def _tc_proj(h, wi, ws, bs):
  return pl.pallas_call(
      _proj_body,
      out_shape=[jax.ShapeDtypeStruct((N_NODES, H), jnp.float32)] * 2,
  )(h, wi, ws, bs.reshape(1, H))


def _degfin_body(dp_ref, recip_ref, scale_ref):
  d = dp_ref[0] + dp_ref[1]
  recip_ref[...] = 1.0 / (d[:, 0:1] + 1.0)
  scale_ref[...] = jnp.log(d[:, 1:2] + 1.0)


def _tc_degfin(deg_p):
  return pl.pallas_call(
      _degfin_body,
      out_shape=[jax.ShapeDtypeStruct((N_NODES, 1), jnp.float32)] * 2,
  )(deg_p)


def _update_body(h_ref, ai_ref, as_ref, vld_ref, vpd_ref, vl_ref, vp_ref,
                 recip_ref, scale_ref, ho_ref, vlo_ref, vpo_ref):
  vl = vl_ref[...] + vld_ref[0] + vld_ref[1]
  vp = vp_ref[...] + vpd_ref[0] + vpd_ref[1]
  vlo_ref[...] = vl
  vpo_ref[...] = vp
  coup = jnp.tanh(jnp.sum(vl * vp, axis=1, keepdims=True))
  aggi = (ai_ref[0] + ai_ref[1]) * recip_ref[...]
  aggs = (as_ref[0] + as_ref[1]) * scale_ref[...]
  ho_ref[...] = h_ref[...] + aggi + aggs + 0.1 * coup


def _tc_update(h, aggi_p, aggs_p, vld_p, vpd_p, vl, vp, recip, scale):
  rb = 2000
  grid = N_NODES // rb
  return pl.pallas_call(
      _update_body,
      grid=(grid,),
      in_specs=[
          pl.BlockSpec((rb, H), lambda i: (i, 0)),
          pl.BlockSpec((NC, rb, H), lambda i: (0, i, 0)),
          pl.BlockSpec((NC, rb, H), lambda i: (0, i, 0)),
          pl.BlockSpec((NC, rb, 4), lambda i: (0, i, 0)),
          pl.BlockSpec((NC, rb, 4), lambda i: (0, i, 0)),
          pl.BlockSpec((rb, 4), lambda i: (i, 0)),
          pl.BlockSpec((rb, 4), lambda i: (i, 0)),
          pl.BlockSpec((rb, 1), lambda i: (i, 0)),
          pl.BlockSpec((rb, 1), lambda i: (i, 0)),
      ],
      out_specs=[
          pl.BlockSpec((rb, H), lambda i: (i, 0)),
          pl.BlockSpec((rb, 4), lambda i: (i, 0)),
          pl.BlockSpec((rb, 4), lambda i: (i, 0)),
      ],
      out_shape=[
          jax.ShapeDtypeStruct((N_NODES, H), jnp.float32),
          jax.ShapeDtypeStruct((N_NODES, 4), jnp.float32),
          jax.ShapeDtypeStruct((N_NODES, 4), jnp.float32),
      ],
  )(h, aggi_p, aggs_p, vld_p, vpd_p, vl, vp, recip, scale)


def _head_body(h_ref, b_ref, fw_ref, fb_ref, gam_ref, bet_ref,
               fwo_ref, fbo_ref, out_ref):
  gid = lax.broadcasted_iota(jnp.int32, (NUM_GRAPHS, 1), 0)
  onehot = (gid == b_ref[...]).astype(jnp.float32)
  g = jnp.dot(onehot, h_ref[...], preferred_element_type=jnp.float32)
  for j in range(3):
    g = jnp.dot(g, fw_ref[j], preferred_element_type=jnp.float32) + fb_ref[j]
    g = jnp.where(g > 0, g, 0.01 * g)
    mu = jnp.mean(g, axis=0)
    d = g - mu
    var = jnp.mean(d * d, axis=0)
    g = gam_ref[j] * d / jnp.sqrt(var + 1e-5) + bet_ref[j]
  out_ref[...] = (jnp.dot(g, fwo_ref[...], preferred_element_type=jnp.float32)
                  + fbo_ref[...])


def _tc_head(h, batch_row, fw_all, fb_all, gam_all, bet_all, fwo, fbo):
  return pl.pallas_call(
      _head_body,
      out_shape=jax.ShapeDtypeStruct((NUM_GRAPHS, 1), jnp.float32),
  )(h, batch_row, fw_all, fb_all, gam_all, bet_all, fwo, fbo.reshape(1, 1))


def _zero_flat(tab, n):
  z = jnp.zeros((LANES,), jnp.float32)
  def body(i, _):
    tab[pl.ds(i * LANES, LANES)] = z
    return 0
  lax.fori_loop(0, n // LANES, body, 0)


def _zero_2d4(tab, nrow):
  z = jnp.zeros((LANES,), jnp.float32)
  rows0 = jnp.arange(LANES, dtype=jnp.int32) // 4
  cols = jnp.arange(LANES, dtype=jnp.int32) % 4
  def body(i, _):
    plsc.store_scatter(tab, [i * 4 + rows0, cols], z)
    return 0
  lax.fori_loop(0, nrow // 4, body, 0)


def _coop_zero(sp_tab, zbuf, s):
  nblk = NROWS // CH
  def body(t, _):
    cid = t * NS + s
    @pl.when(cid < nblk)
    def _():
      pltpu.sync_copy(zbuf, sp_tab.at[pl.ds(cid * CH, CH)])
    return 0
  lax.fori_loop(0, (nblk + NS - 1) // NS, body, 0)


def _sc_prep_body(e_i, e_s, si_hbm, di_hbm, ss_hbm, ds_hbm,
                  px_hbm, py_hbm, pz_hbm,
                  reli_out, rels_out, deg_out,
                  src_v, dst_v, px_v, py_v, pz_v, relbuf, deg1, deg2,
                  degtab_sp):
  s = lax.axis_index("s")
  c = lax.axis_index("c")
  wid = s * NC + c
  lane = jnp.arange(LANES, dtype=jnp.int32)

  pltpu.sync_copy(px_hbm, px_v)
  pltpu.sync_copy(py_hbm, py_v)
  pltpu.sync_copy(pz_hbm, pz_v)

  _zero_flat(relbuf, 4 * CH)
  _zero_2d4(deg1, CH)
  _zero_2d4(deg2, CH)
  _coop_zero(degtab_sp, deg1, s)
  plsc.subcore_barrier()
  ones = jnp.full((LANES,), 1.0, jnp.float32)
  for g in range(CH // LANES):
    rows = g * LANES + lane
    plsc.store_scatter(deg1, [rows, jnp.zeros((LANES,), jnp.int32)], ones)
    plsc.store_scatter(deg2, [rows, jnp.ones((LANES,), jnp.int32)], ones)

  for (sh, dh, e_pad, rel_out, degbuf) in (
      (si_hbm, di_hbm, e_i, reli_out, deg1),
      (ss_hbm, ds_hbm, e_s, rels_out, deg2)):
    ew = e_pad // NW
    def chunk(i, _):
      base = wid * ew + i * CH
      pltpu.sync_copy(sh.at[pl.ds(base, CH)], src_v)
      pltpu.sync_copy(dh.at[pl.ds(base, CH)], dst_v)
      def group(g, _):
        si = src_v[pl.ds(g * LANES, LANES)]
        di = dst_v[pl.ds(g * LANES, LANES)]
        flat0 = (g * LANES + lane) * 4
        for comp, tabv in enumerate((px_v, py_v, pz_v)):
          sv = plsc.load_gather(tabv, [si])
          dv = plsc.load_gather(tabv, [di])
          plsc.store_scatter(relbuf, [flat0 + comp], sv - dv)
        return 0
      lax.fori_loop(0, CH // LANES, group, 0)
      pltpu.sync_copy(relbuf, rel_out.at[pl.ds(base * 4, CH * 4)])
      pltpu.sync_copy(degbuf, degtab_sp.at[dst_v], add=True)
      return 0
    lax.fori_loop(0, ew // CH, chunk, 0)

  plsc.subcore_barrier()
  @pl.when(s == 0)
  def _():
    pltpu.sync_copy(degtab_sp.at[pl.ds(0, N_NODES)],
                    deg_out.at[pl.ds(c * N_NODES, N_NODES)])


def _sc_prep(si_i, di_i, ss_s, ds_s, px, py, pz, e_i, e_s):
  kfn = pl.kernel(
      functools.partial(_sc_prep_body, e_i, e_s),
      out_type=[
          jax.ShapeDtypeStruct((e_i * 4,), jnp.float32),
          jax.ShapeDtypeStruct((e_s * 4,), jnp.float32),
          jax.ShapeDtypeStruct((NC * N_NODES, 4), jnp.float32),
      ],
      mesh=_sc_mesh(),
      compiler_params=_SC_PARAMS,
      scratch_types=[
          pltpu.VMEM((CH,), jnp.int32),
          pltpu.VMEM((CH,), jnp.int32),
          pltpu.VMEM((NROWS,), jnp.float32),
          pltpu.VMEM((NROWS,), jnp.float32),
          pltpu.VMEM((NROWS,), jnp.float32),
          pltpu.VMEM((4 * CH,), jnp.float32),
          pltpu.VMEM((CH, 4), jnp.float32),
          pltpu.VMEM((CH, 4), jnp.float32),
          pltpu.VMEM_SHARED((NROWS, 4), jnp.float32),
      ],
  )
  return kfn(si_i, di_i, ss_s, ds_s, px, py, pz)


def _sc_edge_body(has_ea, e_pad, hw_hbm, ea_hbm, src_hbm, dst_hbm,
                  rel_hbm, gv_hbm,
                  agg_out, vd_out,
                  src_v0, src_v1, dst_v0, dst_v1, rows_v0, rows_v1,
                  ea_v0, ea_v1, relv0, relv1, gl_v, stage,
                  agg_sp, vtab_sp, sem_i0, sem_i1, sem_d0, sem_d1):
  s = lax.axis_index("s")
  c = lax.axis_index("c")
  wid = s * NC + c
  ew = e_pad // NW
  nch = ew // CH
  lane = jnp.arange(LANES, dtype=jnp.int32)
  lane4 = lane // 4
  lmod4 = lane % 4
  bfly = [jnp.arange(LANES, dtype=jnp.int32) ^ sh for sh in (8, 4, 2, 1)]
  src_v = (src_v0, src_v1)
  dst_v = (dst_v0, dst_v1)
  rows_v = (rows_v0, rows_v1)
  ea_v = (ea_v0, ea_v1)
  relv = (relv0, relv1)
  sem_i = (sem_i0, sem_i1)
  sem_d = (sem_d0, sem_d1)

  pltpu.sync_copy(gv_hbm, gl_v)
  gl_regs = [gl_v[pl.ds(j * LANES, LANES)] for j in range(H // LANES)]

  def zrow(i, _):
    for j in range(H // LANES):
      rows_v0[i, pl.ds(j * LANES, LANES)] = jnp.zeros((LANES,), jnp.float32)
    return 0
  lax.fori_loop(0, CH, zrow, 0)
  _zero_2d4(stage, CH)
  _coop_zero(agg_sp, rows_v0, s)
  _coop_zero(vtab_sp, stage, s)
  plsc.subcore_barrier()

  def idx_copies(i, b):
    base = wid * ew + i * CH
    return (
        pltpu.make_async_copy(src_hbm.at[pl.ds(base, CH)], src_v[b],
                              sem_i[b]),
        pltpu.make_async_copy(dst_hbm.at[pl.ds(base, CH)], dst_v[b],
                              sem_i[b]),
    )

  def dat_copies(i, b):
    base = wid * ew + i * CH
    cps = [pltpu.make_async_copy(hw_hbm.at[src_v[b]], rows_v[b], sem_d[b]),
           pltpu.make_async_copy(rel_hbm.at[pl.ds(base * 4, CH * 4)],
                                 relv[b], sem_d[b])]
    if has_ea:
      cps.append(pltpu.make_async_copy(ea_hbm.at[pl.ds(base, CH)], ea_v[b],
                                       sem_d[b]))
    return cps

  def issue(cps):
    for cp in cps:
      cp.start()

  def wait(cps):
    for cp in cps:
      cp.wait()

  def compute(i, b):
    sv, dv, rv, ev, lv = src_v[b], dst_v[b], rows_v[b], ea_v[b], relv[b]

    def group(g, _):
      dots = jnp.zeros((LANES,), jnp.float32)
      for k in range(LANES):
        e = g * LANES + k
        acc = jnp.zeros((LANES,), jnp.float32)
        for j in range(H // LANES):
          r = rv[e, pl.ds(j * LANES, LANES)]
          if has_ea:
            r = r + ev[e, pl.ds(j * LANES, LANES)]
          m = r / (1.0 + jnp.exp(-r))
          rv[e, pl.ds(j * LANES, LANES)] = m
          acc = acc + m * gl_regs[j]
        for perm in bfly:
          acc = acc + acc.at[perm].get(mode='promise_in_bounds')
        dots = jnp.where(lane == k, acc, dots)
      a = jnp.abs(dots)
      t = 1.0 - 2.0 / (jnp.exp(2.0 * a) + 1.0)
      gate = jnp.where(dots < 0.0, -t, t)
      for q in range(4):
        e0 = g * LANES + q * 4
        rvv = lv[pl.ds(e0 * 4, LANES)]
        g4 = gate.at[q * 4 + lane4].get(mode='promise_in_bounds')
        plsc.store_scatter(stage, [e0 + lane4, lmod4], rvv * g4)
      return 0

    lax.fori_loop(0, CH // LANES, group, 0)
    pltpu.sync_copy(rv, agg_sp.at[dv], add=True)
    pltpu.sync_copy(stage, vtab_sp.at[dv], add=True)

  ic = idx_copies(0, 0)
  issue(ic)
  wait(ic)
  issue(dat_copies(0, 0))
  if nch > 1:
    issue(idx_copies(1, 1))

  def step(i, b):
    @pl.when(i + 1 < nch)
    def _():
      wait(idx_copies(i + 1, 1 - b))
      issue(dat_copies(i + 1, 1 - b))
    wait(dat_copies(i, b))
    compute(i, b)
    @pl.when(i + 2 < nch)
    def _():
      issue(idx_copies(i + 2, b))

  def pair(j, _):
    step(2 * j, 0)
    step(2 * j + 1, 1)
    return 0
  lax.fori_loop(0, nch // 2, pair, 0)
  if nch % 2:
    step(nch - 1, 0)

  plsc.subcore_barrier()

  @pl.when(s == 0)
  def _():
    pltpu.sync_copy(agg_sp.at[pl.ds(0, N_NODES)],
                    agg_out.at[pl.ds(c * N_NODES, N_NODES)])
    pltpu.sync_copy(vtab_sp.at[pl.ds(0, N_NODES)],
                    vd_out.at[pl.ds(c * N_NODES, N_NODES)])


def _sc_edge_phase(hw, ea, src_pad, dst_pad, rel, gv, e_pad, has_ea):
  ea_shape = (CH, H) if has_ea else (LANES,)
  scratch = [
      pltpu.VMEM((CH,), jnp.int32),
      pltpu.VMEM((CH,), jnp.int32),
      pltpu.VMEM((CH,), jnp.int32),
      pltpu.VMEM((CH,), jnp.int32),
      pltpu.VMEM((CH, H), jnp.float32),
      pltpu.VMEM((CH, H), jnp.float32),
      pltpu.VMEM(ea_shape, jnp.float32),
      pltpu.VMEM(ea_shape, jnp.float32),
      pltpu.VMEM((4 * CH,), jnp.float32),
      pltpu.VMEM((4 * CH,), jnp.float32),
      pltpu.VMEM((H,), jnp.float32),
      pltpu.VMEM((CH, 4), jnp.float32),
      pltpu.VMEM_SHARED((NROWS, H), jnp.float32),
      pltpu.VMEM_SHARED((NROWS, 4), jnp.float32),
      pltpu.SemaphoreType.DMA,
      pltpu.SemaphoreType.DMA,
      pltpu.SemaphoreType.DMA,
      pltpu.SemaphoreType.DMA,
  ]
  out_type = [
      jax.ShapeDtypeStruct((NC * N_NODES, H), jnp.float32),
      jax.ShapeDtypeStruct((NC * N_NODES, 4), jnp.float32),
  ]
  kfn = pl.kernel(
      functools.partial(_sc_edge_body, has_ea, e_pad),
      out_type=out_type,
      mesh=_sc_mesh(),
      compiler_params=_SC_PARAMS,
      scratch_types=scratch,
  )
  if not has_ea:
    ea = jnp.zeros((LANES,), jnp.float32)
  return kfn(hw, ea, src_pad, dst_pad, rel, gv)


def kernel(x, edge_index_intra, edge_index_inter, pos, edge_attr, batch,
           params):
  p = params
  e_i = _epad(edge_index_intra.shape[1])
  e_s = _epad(edge_index_inter.shape[1])
  si_i, di_i = _pad_edges(edge_index_intra[0], edge_index_intra[1], e_i)
  ss_s, ds_s = _pad_edges(edge_index_inter[0], edge_index_inter[1], e_s)
  ea_pad = jnp.concatenate(
      [edge_attr,
       jnp.zeros((e_i - edge_attr.shape[0], D_EDGE), jnp.float32)])
  posp = jnp.concatenate([pos, jnp.zeros((NROWS - N_NODES, 3), jnp.float32)])
  px = posp[:, 0].copy()
  py = posp[:, 1].copy()
  pz = posp[:, 2].copy()

  we_all = jnp.stack([p['We%d' % l] for l in range(4)])
  bi_all = jnp.stack([p['bi%d' % l].reshape(1, H) for l in range(4)])
  eab = _tc_edgefeat(ea_pad, we_all, bi_all, e_i)

  h = _tc_embed(x, p['Wn'], p['bn'])

  rel_i, rel_s, deg_p = _sc_prep(si_i, di_i, ss_s, ds_s, px, py, pz, e_i, e_s)
  recip, scale = _tc_degfin(deg_p.reshape(NC, N_NODES, 4))

  vl = jnp.zeros((N_NODES, 4), jnp.float32)
  vp = jnp.zeros((N_NODES, 4), jnp.float32)
  for l in range(4):
    hwi, hws = _tc_proj(h, p['Wi%d' % l], p['Ws%d' % l], p['bs%d' % l])
    aggi_p, vld_p = _sc_edge_phase(
        hwi, eab[l], si_i, di_i, rel_i, p['gl%d' % l], e_i, True)
    aggs_p, vpd_p = _sc_edge_phase(
        hws, None, ss_s, ds_s, rel_s, p['gp%d' % l], e_s, False)
    h, vl, vp = _tc_update(h, aggi_p.reshape(NC, N_NODES, H),
                           aggs_p.reshape(NC, N_NODES, H),
                           vld_p.reshape(NC, N_NODES, 4),
                           vpd_p.reshape(NC, N_NODES, 4), vl, vp,
                           recip, scale)

  fw_all = jnp.stack([p['fW%d' % j] for j in range(3)])
  fb_all = jnp.stack([p['fb%d' % j].reshape(1, H) for j in range(3)])
  gam_all = jnp.stack([p['gamma%d' % j].reshape(1, H) for j in range(3)])
  bet_all = jnp.stack([p['beta%d' % j].reshape(1, H) for j in range(3)])
  out = _tc_head(h, batch.reshape(1, N_NODES), fw_all, fb_all, gam_all,
                 bet_all, p['fWout'], p['fbout'])
  return out.reshape(-1)


# speedup vs baseline: 1.0021x; 1.0021x over previous
import functools
import jax
import jax.numpy as jnp
from jax import lax
from jax.experimental import pallas as pl
from jax.experimental.pallas import tpu as pltpu
from jax.experimental.pallas import tpu_sc as plsc

N_NODES = 10000
D_NODE = 128
D_EDGE = 16
H = 128
NUM_GRAPHS = 64

NC = 2
NS = 16
NW = NC * NS
CH = 96
LANES = 16

NROWS = 10048
DUMMY = N_NODES

_SC_PARAMS = pltpu.CompilerParams(
    needs_layout_passes=False, use_tc_tiling_on_sc=False)


def _sc_mesh():
  return plsc.VectorSubcoreMesh(core_axis_name="c", subcore_axis_name="s",
                                num_cores=NC, num_subcores=NS)


def _pad_edges(e_src, e_dst, n_pad):
  pe = n_pad - e_src.shape[0]
  src = jnp.concatenate([e_src, jnp.zeros((pe,), jnp.int32)])
  dst = jnp.concatenate([e_dst, jnp.full((pe,), DUMMY, jnp.int32)])
  return src, dst


def _epad(e):
  per = NW * CH
  return ((e + per - 1) // per) * per


def _embed_body(x_ref, wn_ref, bn_ref, out_ref):
  z = jnp.dot(x_ref[...], wn_ref[...], preferred_element_type=jnp.float32)
  z = z + bn_ref[...]
  out_ref[...] = z / (1.0 + jnp.exp(-z))


def _tc_embed(x, wn, bn):
  return pl.pallas_call(
      _embed_body,
      out_shape=jax.ShapeDtypeStruct((N_NODES, H), jnp.float32),
  )(x, wn, bn.reshape(1, H))


def _edgefeat_body(ea_ref, we_ref, bi_ref, o0, o1, o2, o3):
  ea = ea_ref[...]
  outs = (o0, o1, o2, o3)
  for l in range(4):
    z = jnp.dot(ea, we_ref[l], preferred_element_type=jnp.float32)
    outs[l][...] = z + bi_ref[l]


def _tc_edgefeat(ea_pad, we_all, bi_all, e_pad):
  blk = 4096
  grid = e_pad // blk
  outs = [jax.ShapeDtypeStruct((e_pad, H), jnp.float32)] * 4
  return pl.pallas_call(
      _edgefeat_body,
      grid=(grid,),
      in_specs=[
          pl.BlockSpec((blk, D_EDGE), lambda i: (i, 0)),
          pl.BlockSpec((4, D_EDGE, H), lambda i: (0, 0, 0)),
          pl.BlockSpec((4, 1, H), lambda i: (0, 0, 0)),
      ],
      out_specs=[pl.BlockSpec((blk, H), lambda i: (i, 0))] * 4,
      out_shape=outs,
  )(ea_pad, we_all, bi_all)


def _proj_body(h_ref, wi_ref, ws_ref, bs_ref, oi_ref, os_ref):
  h = h_ref[...]
  oi_ref[...] = jnp.dot(h, wi_ref[...], preferred_element_type=jnp.float32)
  os_ref[...] = (jnp.dot(h, ws_ref[...], preferred_element_type=jnp.float32)
                 + bs_ref[...])


def _tc_proj(h, wi, ws, bs):
  return pl.pallas_call(
      _proj_body,
      out_shape=[jax.ShapeDtypeStruct((N_NODES, H), jnp.float32)] * 2,
  )(h, wi, ws, bs.reshape(1, H))


def _degfin_body(dp_ref, recip_ref, scale_ref):
  d = dp_ref[0] + dp_ref[1]
  recip_ref[...] = 1.0 / (d[:, 0:1] + 1.0)
  scale_ref[...] = jnp.log(d[:, 1:2] + 1.0)


def _tc_degfin(deg_p):
  return pl.pallas_call(
      _degfin_body,
      out_shape=[jax.ShapeDtypeStruct((N_NODES, 1), jnp.float32)] * 2,
  )(deg_p)


def _update_body(h_ref, ai_ref, as_ref, vld_ref, vpd_ref, vl_ref, vp_ref,
                 recip_ref, scale_ref, ho_ref, vlo_ref, vpo_ref):
  vl = vl_ref[...] + vld_ref[0] + vld_ref[1]
  vp = vp_ref[...] + vpd_ref[0] + vpd_ref[1]
  vlo_ref[...] = vl
  vpo_ref[...] = vp
  coup = jnp.tanh(jnp.sum(vl * vp, axis=1, keepdims=True))
  aggi = (ai_ref[0] + ai_ref[1]) * recip_ref[...]
  aggs = (as_ref[0] + as_ref[1]) * scale_ref[...]
  ho_ref[...] = h_ref[...] + aggi + aggs + 0.1 * coup


def _tc_update(h, aggi_p, aggs_p, vld_p, vpd_p, vl, vp, recip, scale):
  rb = 2000
  grid = N_NODES // rb
  return pl.pallas_call(
      _update_body,
      grid=(grid,),
      in_specs=[
          pl.BlockSpec((rb, H), lambda i: (i, 0)),
          pl.BlockSpec((NC, rb, H), lambda i: (0, i, 0)),
          pl.BlockSpec((NC, rb, H), lambda i: (0, i, 0)),
          pl.BlockSpec((NC, rb, 4), lambda i: (0, i, 0)),
          pl.BlockSpec((NC, rb, 4), lambda i: (0, i, 0)),
          pl.BlockSpec((rb, 4), lambda i: (i, 0)),
          pl.BlockSpec((rb, 4), lambda i: (i, 0)),
          pl.BlockSpec((rb, 1), lambda i: (i, 0)),
          pl.BlockSpec((rb, 1), lambda i: (i, 0)),
      ],
      out_specs=[
          pl.BlockSpec((rb, H), lambda i: (i, 0)),
          pl.BlockSpec((rb, 4), lambda i: (i, 0)),
          pl.BlockSpec((rb, 4), lambda i: (i, 0)),
      ],
      out_shape=[
          jax.ShapeDtypeStruct((N_NODES, H), jnp.float32),
          jax.ShapeDtypeStruct((N_NODES, 4), jnp.float32),
          jax.ShapeDtypeStruct((N_NODES, 4), jnp.float32),
      ],
  )(h, aggi_p, aggs_p, vld_p, vpd_p, vl, vp, recip, scale)


def _head_body(h_ref, b_ref, fw_ref, fb_ref, gam_ref, bet_ref,
               fwo_ref, fbo_ref, out_ref):
  gid = lax.broadcasted_iota(jnp.int32, (NUM_GRAPHS, 1), 0)
  onehot = (gid == b_ref[...]).astype(jnp.float32)
  g = jnp.dot(onehot, h_ref[...], preferred_element_type=jnp.float32)
  for j in range(3):
    g = jnp.dot(g, fw_ref[j], preferred_element_type=jnp.float32) + fb_ref[j]
    g = jnp.where(g > 0, g, 0.01 * g)
    mu = jnp.mean(g, axis=0)
    d = g - mu
    var = jnp.mean(d * d, axis=0)
    g = gam_ref[j] * d / jnp.sqrt(var + 1e-5) + bet_ref[j]
  out_ref[...] = (jnp.dot(g, fwo_ref[...], preferred_element_type=jnp.float32)
                  + fbo_ref[...])


def _tc_head(h, batch_row, fw_all, fb_all, gam_all, bet_all, fwo, fbo):
  return pl.pallas_call(
      _head_body,
      out_shape=jax.ShapeDtypeStruct((NUM_GRAPHS, 1), jnp.float32),
  )(h, batch_row, fw_all, fb_all, gam_all, bet_all, fwo, fbo.reshape(1, 1))


def _zero_flat(tab, n):
  z = jnp.zeros((LANES,), jnp.float32)
  def body(i, _):
    tab[pl.ds(i * LANES, LANES)] = z
    return 0
  lax.fori_loop(0, n // LANES, body, 0)


def _zero_2d4(tab, nrow):
  z = jnp.zeros((LANES,), jnp.float32)
  rows0 = jnp.arange(LANES, dtype=jnp.int32) // 4
  cols = jnp.arange(LANES, dtype=jnp.int32) % 4
  def body(i, _):
    plsc.store_scatter(tab, [i * 4 + rows0, cols], z)
    return 0
  lax.fori_loop(0, nrow // 4, body, 0)


def _coop_zero(sp_tab, zbuf, s):
  nblk = NROWS // 64
  def body(t, _):
    cid = t * NS + s
    @pl.when(cid < nblk)
    def _():
      pltpu.sync_copy(zbuf.at[pl.ds(0, 64)], sp_tab.at[pl.ds(cid * 64, 64)])
    return 0
  lax.fori_loop(0, (nblk + NS - 1) // NS, body, 0)


def _sc_prep_body(e_i, e_s, si_hbm, di_hbm, ss_hbm, ds_hbm,
                  px_hbm, py_hbm, pz_hbm,
                  reli_out, rels_out, deg_out,
                  src_v, dst_v, px_v, py_v, pz_v, relbuf, deg1, deg2,
                  degtab_sp):
  s = lax.axis_index("s")
  c = lax.axis_index("c")
  wid = s * NC + c
  lane = jnp.arange(LANES, dtype=jnp.int32)

  pltpu.sync_copy(px_hbm, px_v)
  pltpu.sync_copy(py_hbm, py_v)
  pltpu.sync_copy(pz_hbm, pz_v)

  _zero_flat(relbuf, 4 * CH)
  _zero_2d4(deg1, CH)
  _zero_2d4(deg2, CH)
  _coop_zero(degtab_sp, deg1, s)
  plsc.subcore_barrier()
  ones = jnp.full((LANES,), 1.0, jnp.float32)
  for g in range(CH // LANES):
    rows = g * LANES + lane
    plsc.store_scatter(deg1, [rows, jnp.zeros((LANES,), jnp.int32)], ones)
    plsc.store_scatter(deg2, [rows, jnp.ones((LANES,), jnp.int32)], ones)

  for (sh, dh, e_pad, rel_out, degbuf) in (
      (si_hbm, di_hbm, e_i, reli_out, deg1),
      (ss_hbm, ds_hbm, e_s, rels_out, deg2)):
    ew = e_pad // NW
    def chunk(i, _):
      base = wid * ew + i * CH
      pltpu.sync_copy(sh.at[pl.ds(base, CH)], src_v)
      pltpu.sync_copy(dh.at[pl.ds(base, CH)], dst_v)
      def group(g, _):
        si = src_v[pl.ds(g * LANES, LANES)]
        di = dst_v[pl.ds(g * LANES, LANES)]
        flat0 = (g * LANES + lane) * 4
        for comp, tabv in enumerate((px_v, py_v, pz_v)):
          sv = plsc.load_gather(tabv, [si])
          dv = plsc.load_gather(tabv, [di])
          plsc.store_scatter(relbuf, [flat0 + comp], sv - dv)
        return 0
      lax.fori_loop(0, CH // LANES, group, 0)
      pltpu.sync_copy(relbuf, rel_out.at[pl.ds(base * 4, CH * 4)])
      pltpu.sync_copy(degbuf, degtab_sp.at[dst_v], add=True)
      return 0
    lax.fori_loop(0, ew // CH, chunk, 0)

  plsc.subcore_barrier()
  @pl.when(s == 0)
  def _():
    pltpu.sync_copy(degtab_sp.at[pl.ds(0, N_NODES)],
                    deg_out.at[pl.ds(c * N_NODES, N_NODES)])


def _sc_prep(si_i, di_i, ss_s, ds_s, px, py, pz, e_i, e_s):
  kfn = pl.kernel(
      functools.partial(_sc_prep_body, e_i, e_s),
      out_type=[
          jax.ShapeDtypeStruct((e_i * 4,), jnp.float32),
          jax.ShapeDtypeStruct((e_s * 4,), jnp.float32),
          jax.ShapeDtypeStruct((NC * N_NODES, 4), jnp.float32),
      ],
      mesh=_sc_mesh(),
      compiler_params=_SC_PARAMS,
      scratch_types=[
          pltpu.VMEM((CH,), jnp.int32),
          pltpu.VMEM((CH,), jnp.int32),
          pltpu.VMEM((NROWS,), jnp.float32),
          pltpu.VMEM((NROWS,), jnp.float32),
          pltpu.VMEM((NROWS,), jnp.float32),
          pltpu.VMEM((4 * CH,), jnp.float32),
          pltpu.VMEM((CH, 4), jnp.float32),
          pltpu.VMEM((CH, 4), jnp.float32),
          pltpu.VMEM_SHARED((NROWS, 4), jnp.float32),
      ],
  )
  return kfn(si_i, di_i, ss_s, ds_s, px, py, pz)


def _sc_edge_body(has_ea, e_pad, hw_hbm, ea_hbm, src_hbm, dst_hbm,
                  rel_hbm, gv_hbm,
                  agg_out, vd_out,
                  src_v0, src_v1, dst_v, rows_v0, rows_v1,
                  ea_v, relv, gl_v, stage,
                  agg_sp, vtab_sp, sem_s0, sem_s1, sem_g0, sem_g1, sem_m):
  s = lax.axis_index("s")
  c = lax.axis_index("c")
  wid = s * NC + c
  ew = e_pad // NW
  nch = ew // CH
  lane = jnp.arange(LANES, dtype=jnp.int32)
  lane4 = lane // 4
  lmod4 = lane % 4
  bfly = [jnp.arange(LANES, dtype=jnp.int32) ^ sh for sh in (8, 4, 2, 1)]
  src_v = (src_v0, src_v1)
  rows_v = (rows_v0, rows_v1)
  sem_s = (sem_s0, sem_s1)
  sem_g = (sem_g0, sem_g1)

  pltpu.sync_copy(gv_hbm, gl_v)
  gl_regs = [gl_v[pl.ds(j * LANES, LANES)] for j in range(H // LANES)]

  def zrow(i, _):
    for j in range(H // LANES):
      rows_v0[i, pl.ds(j * LANES, LANES)] = jnp.zeros((LANES,), jnp.float32)
    return 0
  lax.fori_loop(0, CH, zrow, 0)
  _zero_2d4(stage, CH)
  _coop_zero(agg_sp, rows_v0, s)
  _coop_zero(vtab_sp, stage, s)
  plsc.subcore_barrier()

  def srcidx_cp(i, b):
    base = wid * ew + i * CH
    return pltpu.make_async_copy(src_hbm.at[pl.ds(base, CH)], src_v[b],
                                 sem_s[b])

  def gather_cp(i, b):
    return pltpu.make_async_copy(hw_hbm.at[src_v[b]], rows_v[b], sem_g[b])

  def misc_cps(i):
    base = wid * ew + i * CH
    cps = [pltpu.make_async_copy(dst_hbm.at[pl.ds(base, CH)], dst_v, sem_m),
           pltpu.make_async_copy(rel_hbm.at[pl.ds(base * 4, CH * 4)],
                                 relv, sem_m)]
    if has_ea:
      cps.append(pltpu.make_async_copy(ea_hbm.at[pl.ds(base, CH)], ea_v,
                                       sem_m))
    return cps

  def compute(i, b):
    rv = rows_v[b]

    def group(g, _):
      dots = jnp.zeros((LANES,), jnp.float32)
      for k in range(LANES):
        e = g * LANES + k
        acc = jnp.zeros((LANES,), jnp.float32)
        for j in range(H // LANES):
          r = rv[e, pl.ds(j * LANES, LANES)]
          if has_ea:
            r = r + ea_v[e, pl.ds(j * LANES, LANES)]
          m = r / (1.0 + jnp.exp(-r))
          rv[e, pl.ds(j * LANES, LANES)] = m
          acc = acc + m * gl_regs[j]
        for perm in bfly:
          acc = acc + acc.at[perm].get(mode='promise_in_bounds')
        dots = jnp.where(lane == k, acc, dots)
      a = jnp.abs(dots)
      t = 1.0 - 2.0 / (jnp.exp(2.0 * a) + 1.0)
      gate = jnp.where(dots < 0.0, -t, t)
      for q in range(4):
        e0 = g * LANES + q * 4
        rvv = relv[pl.ds(e0 * 4, LANES)]
        g4 = gate.at[q * 4 + lane4].get(mode='promise_in_bounds')
        plsc.store_scatter(stage, [e0 + lane4, lmod4], rvv * g4)
      return 0

    lax.fori_loop(0, CH // LANES, group, 0)
    pltpu.sync_copy(rv, agg_sp.at[dst_v], add=True)
    pltpu.sync_copy(stage, vtab_sp.at[dst_v], add=True)

  srcidx_cp(0, 0).start()
  srcidx_cp(0, 0).wait()
  gather_cp(0, 0).start()
  if nch > 1:
    srcidx_cp(1, 1).start()
  for cp in misc_cps(0):
    cp.start()
    cp.wait()

  def step(i, b):
    @pl.when(i + 1 < nch)
    def _():
      srcidx_cp(i + 1, 1 - b).wait()
    gather_cp(i, b).wait()
    @pl.when(i + 1 < nch)
    def _():
      gather_cp(i + 1, 1 - b).start()
    @pl.when(i + 2 < nch)
    def _():
      srcidx_cp(i + 2, b).start()
    @pl.when(i > 0)
    def _():
      for cp in misc_cps(i):
        cp.wait()
    compute(i, b)
    @pl.when(i + 1 < nch)
    def _():
      for cp in misc_cps(i + 1):
        cp.start()

  def pair(j, _):
    step(2 * j, 0)
    step(2 * j + 1, 1)
    return 0
  lax.fori_loop(0, nch // 2, pair, 0)
  if nch % 2:
    step(nch - 1, 0)

  plsc.subcore_barrier()

  @pl.when(s == 0)
  def _():
    pltpu.sync_copy(agg_sp.at[pl.ds(0, N_NODES)],
                    agg_out.at[pl.ds(c * N_NODES, N_NODES)])
    pltpu.sync_copy(vtab_sp.at[pl.ds(0, N_NODES)],
                    vd_out.at[pl.ds(c * N_NODES, N_NODES)])


def _sc_edge_phase(hw, ea, src_pad, dst_pad, rel, gv, e_pad, has_ea):
  ea_shape = (CH, H) if has_ea else (LANES,)
  scratch = [
      pltpu.VMEM((CH,), jnp.int32),
      pltpu.VMEM((CH,), jnp.int32),
      pltpu.VMEM((CH,), jnp.int32),
      pltpu.VMEM((CH, H), jnp.float32),
      pltpu.VMEM((CH, H), jnp.float32),
      pltpu.VMEM(ea_shape, jnp.float32),
      pltpu.VMEM((4 * CH,), jnp.float32),
      pltpu.VMEM((H,), jnp.float32),
      pltpu.VMEM((CH, 4), jnp.float32),
      pltpu.VMEM_SHARED((NROWS, H), jnp.float32),
      pltpu.VMEM_SHARED((NROWS, 4), jnp.float32),
      pltpu.SemaphoreType.DMA,
      pltpu.SemaphoreType.DMA,
      pltpu.SemaphoreType.DMA,
      pltpu.SemaphoreType.DMA,
      pltpu.SemaphoreType.DMA,
  ]
  out_type = [
      jax.ShapeDtypeStruct((NC * N_NODES, H), jnp.float32),
      jax.ShapeDtypeStruct((NC * N_NODES, 4), jnp.float32),
  ]
  kfn = pl.kernel(
      functools.partial(_sc_edge_body, has_ea, e_pad),
      out_type=out_type,
      mesh=_sc_mesh(),
      compiler_params=_SC_PARAMS,
      scratch_types=scratch,
  )
  if not has_ea:
    ea = jnp.zeros((LANES,), jnp.float32)
  return kfn(hw, ea, src_pad, dst_pad, rel, gv)


def kernel(x, edge_index_intra, edge_index_inter, pos, edge_attr, batch,
           params):
  p = params
  e_i = _epad(edge_index_intra.shape[1])
  e_s = _epad(edge_index_inter.shape[1])
  si_i, di_i = _pad_edges(edge_index_intra[0], edge_index_intra[1], e_i)
  ss_s, ds_s = _pad_edges(edge_index_inter[0], edge_index_inter[1], e_s)
  ea_pad = jnp.concatenate(
      [edge_attr,
       jnp.zeros((e_i - edge_attr.shape[0], D_EDGE), jnp.float32)])
  posp = jnp.concatenate([pos, jnp.zeros((NROWS - N_NODES, 3), jnp.float32)])
  px = posp[:, 0].copy()
  py = posp[:, 1].copy()
  pz = posp[:, 2].copy()

  we_all = jnp.stack([p['We%d' % l] for l in range(4)])
  bi_all = jnp.stack([p['bi%d' % l].reshape(1, H) for l in range(4)])
  eab = _tc_edgefeat(ea_pad, we_all, bi_all, e_i)

  h = _tc_embed(x, p['Wn'], p['bn'])

  rel_i, rel_s, deg_p = _sc_prep(si_i, di_i, ss_s, ds_s, px, py, pz, e_i, e_s)
  recip, scale = _tc_degfin(deg_p.reshape(NC, N_NODES, 4))

  vl = jnp.zeros((N_NODES, 4), jnp.float32)
  vp = jnp.zeros((N_NODES, 4), jnp.float32)
  for l in range(4):
    hwi, hws = _tc_proj(h, p['Wi%d' % l], p['Ws%d' % l], p['bs%d' % l])
    aggi_p, vld_p = _sc_edge_phase(
        hwi, eab[l], si_i, di_i, rel_i, p['gl%d' % l], e_i, True)
    aggs_p, vpd_p = _sc_edge_phase(
        hws, None, ss_s, ds_s, rel_s, p['gp%d' % l], e_s, False)
    h, vl, vp = _tc_update(h, aggi_p.reshape(NC, N_NODES, H),
                           aggs_p.reshape(NC, N_NODES, H),
                           vld_p.reshape(NC, N_NODES, 4),
                           vpd_p.reshape(NC, N_NODES, 4), vl, vp,
                           recip, scale)

  fw_all = jnp.stack([p['fW%d' % j] for j in range(3)])
  fb_all = jnp.stack([p['fb%d' % j].reshape(1, H) for j in range(3)])
  gam_all = jnp.stack([p['gamma%d' % j].reshape(1, H) for j in range(3)])
  bet_all = jnp.stack([p['beta%d' % j].reshape(1, H) for j in range(3)])
  out = _tc_head(h, batch.reshape(1, N_NODES), fw_all, fb_all, gam_all,
                 bet_all, p['fWout'], p['fbout'])
  return out.reshape(-1)


# speedup vs baseline: 1.0552x; 1.0531x over previous
import functools
import jax
import jax.numpy as jnp
from jax import lax
from jax.experimental import pallas as pl
from jax.experimental.pallas import tpu as pltpu
from jax.experimental.pallas import tpu_sc as plsc

N_NODES = 10000
D_NODE = 128
D_EDGE = 16
H = 128
NUM_GRAPHS = 64

NC = 2
NS = 16
NW = NC * NS
CH = 96
LANES = 16

NROWS = 10048
DUMMY = N_NODES

_SC_PARAMS = pltpu.CompilerParams(
    needs_layout_passes=False, use_tc_tiling_on_sc=False)


def _sc_mesh():
  return plsc.VectorSubcoreMesh(core_axis_name="c", subcore_axis_name="s",
                                num_cores=NC, num_subcores=NS)


def _pad_edges(e_src, e_dst, n_pad):
  pe = n_pad - e_src.shape[0]
  src = jnp.concatenate([e_src, jnp.zeros((pe,), jnp.int32)])
  dst = jnp.concatenate([e_dst, jnp.full((pe,), DUMMY, jnp.int32)])
  return src, dst


def _epad(e):
  per = NW * CH
  return ((e + per - 1) // per) * per


def _embed_body(x_ref, wn_ref, bn_ref, out_ref):
  z = jnp.dot(x_ref[...], wn_ref[...], preferred_element_type=jnp.float32)
  z = z + bn_ref[...]
  out_ref[...] = z / (1.0 + jnp.exp(-z))


def _tc_embed(x, wn, bn):
  return pl.pallas_call(
      _embed_body,
      out_shape=jax.ShapeDtypeStruct((N_NODES, H), jnp.float32),
  )(x, wn, bn.reshape(1, H))


def _edgefeat_body(ea_ref, we_ref, bi_ref, o0, o1, o2, o3):
  ea = ea_ref[...]
  outs = (o0, o1, o2, o3)
  for l in range(4):
    z = jnp.dot(ea, we_ref[l], preferred_element_type=jnp.float32)
    outs[l][...] = z + bi_ref[l]


def _tc_edgefeat(ea_pad, we_all, bi_all, e_pad):
  blk = NW * CH
  grid = e_pad // blk
  outs = [jax.ShapeDtypeStruct((e_pad, H), jnp.float32)] * 4
  return pl.pallas_call(
      _edgefeat_body,
      grid=(grid,),
      in_specs=[
          pl.BlockSpec((blk, D_EDGE), lambda i: (i, 0)),
          pl.BlockSpec((4, D_EDGE, H), lambda i: (0, 0, 0)),
          pl.BlockSpec((4, 1, H), lambda i: (0, 0, 0)),
      ],
      out_specs=[pl.BlockSpec((blk, H), lambda i: (i, 0))] * 4,
      out_shape=outs,
  )(ea_pad, we_all, bi_all)


def _proj_body(h_ref, wi_ref, ws_ref, bs_ref, oi_ref, os_ref):
  h = h_ref[...]
  oi_ref[...] = jnp.dot(h, wi_ref[...], preferred_element_type=jnp.float32)
  os_ref[...] = (jnp.dot(h, ws_ref[...], preferred_element_type=jnp.float32)
                 + bs_ref[...])


def _tc_proj(h, wi, ws, bs):
  return pl.pallas_call(
      _proj_body,
      out_shape=[jax.ShapeDtypeStruct((N_NODES, H), jnp.float32)] * 2,
  )(h, wi, ws, bs.reshape(1, H))


def _degfin_body(dp_ref, recip_ref, scale_ref):
  d = dp_ref[0] + dp_ref[1]
  recip_ref[...] = 1.0 / (d[:, 0:1] + 1.0)
  scale_ref[...] = jnp.log(d[:, 1:2] + 1.0)


def _tc_degfin(deg_p):
  return pl.pallas_call(
      _degfin_body,
      out_shape=[jax.ShapeDtypeStruct((N_NODES, 1), jnp.float32)] * 2,
  )(deg_p)


def _update_body(h_ref, ai_ref, as_ref, vld_ref, vpd_ref, vl_ref, vp_ref,
                 recip_ref, scale_ref, ho_ref, vlo_ref, vpo_ref):
  vl = vl_ref[...] + vld_ref[0] + vld_ref[1]
  vp = vp_ref[...] + vpd_ref[0] + vpd_ref[1]
  vlo_ref[...] = vl
  vpo_ref[...] = vp
  coup = jnp.tanh(jnp.sum(vl * vp, axis=1, keepdims=True))
  aggi = (ai_ref[0] + ai_ref[1]) * recip_ref[...]
  aggs = (as_ref[0] + as_ref[1]) * scale_ref[...]
  ho_ref[...] = h_ref[...] + aggi + aggs + 0.1 * coup


def _tc_update(h, aggi_p, aggs_p, vld_p, vpd_p, vl, vp, recip, scale):
  rb = 2000
  grid = N_NODES // rb
  return pl.pallas_call(
      _update_body,
      grid=(grid,),
      in_specs=[
          pl.BlockSpec((rb, H), lambda i: (i, 0)),
          pl.BlockSpec((NC, rb, H), lambda i: (0, i, 0)),
          pl.BlockSpec((NC, rb, H), lambda i: (0, i, 0)),
          pl.BlockSpec((NC, rb, 4), lambda i: (0, i, 0)),
          pl.BlockSpec((NC, rb, 4), lambda i: (0, i, 0)),
          pl.BlockSpec((rb, 4), lambda i: (i, 0)),
          pl.BlockSpec((rb, 4), lambda i: (i, 0)),
          pl.BlockSpec((rb, 1), lambda i: (i, 0)),
          pl.BlockSpec((rb, 1), lambda i: (i, 0)),
      ],
      out_specs=[
          pl.BlockSpec((rb, H), lambda i: (i, 0)),
          pl.BlockSpec((rb, 4), lambda i: (i, 0)),
          pl.BlockSpec((rb, 4), lambda i: (i, 0)),
      ],
      out_shape=[
          jax.ShapeDtypeStruct((N_NODES, H), jnp.float32),
          jax.ShapeDtypeStruct((N_NODES, 4), jnp.float32),
          jax.ShapeDtypeStruct((N_NODES, 4), jnp.float32),
      ],
  )(h, aggi_p, aggs_p, vld_p, vpd_p, vl, vp, recip, scale)


def _head_body(h_ref, b_ref, fw_ref, fb_ref, gam_ref, bet_ref,
               fwo_ref, fbo_ref, out_ref):
  gid = lax.broadcasted_iota(jnp.int32, (NUM_GRAPHS, 1), 0)
  onehot = (gid == b_ref[...]).astype(jnp.float32)
  g = jnp.dot(onehot, h_ref[...], preferred_element_type=jnp.float32)
  for j in range(3):
    g = jnp.dot(g, fw_ref[j], preferred_element_type=jnp.float32) + fb_ref[j]
    g = jnp.where(g > 0, g, 0.01 * g)
    mu = jnp.mean(g, axis=0)
    d = g - mu
    var = jnp.mean(d * d, axis=0)
    g = gam_ref[j] * d / jnp.sqrt(var + 1e-5) + bet_ref[j]
  out_ref[...] = (jnp.dot(g, fwo_ref[...], preferred_element_type=jnp.float32)
                  + fbo_ref[...])


def _tc_head(h, batch_row, fw_all, fb_all, gam_all, bet_all, fwo, fbo):
  return pl.pallas_call(
      _head_body,
      out_shape=jax.ShapeDtypeStruct((NUM_GRAPHS, 1), jnp.float32),
  )(h, batch_row, fw_all, fb_all, gam_all, bet_all, fwo, fbo.reshape(1, 1))


def _zero_flat(tab, n):
  z = jnp.zeros((LANES,), jnp.float32)
  def body(i, _):
    tab[pl.ds(i * LANES, LANES)] = z
    return 0
  lax.fori_loop(0, n // LANES, body, 0)


def _zero_2d4(tab, nrow):
  z = jnp.zeros((LANES,), jnp.float32)
  rows0 = jnp.arange(LANES, dtype=jnp.int32) // 4
  cols = jnp.arange(LANES, dtype=jnp.int32) % 4
  def body(i, _):
    plsc.store_scatter(tab, [i * 4 + rows0, cols], z)
    return 0
  lax.fori_loop(0, nrow // 4, body, 0)


def _coop_zero(sp_tab, zbuf, s):
  nblk = NROWS // 64
  def body(t, _):
    cid = t * NS + s
    @pl.when(cid < nblk)
    def _():
      pltpu.sync_copy(zbuf.at[pl.ds(0, 64)], sp_tab.at[pl.ds(cid * 64, 64)])
    return 0
  lax.fori_loop(0, (nblk + NS - 1) // NS, body, 0)


def _sc_prep_body(e_i, e_s, si_hbm, di_hbm, ss_hbm, ds_hbm,
                  px_hbm, py_hbm, pz_hbm,
                  reli_out, rels_out, deg_out,
                  src_v, dst_v, px_v, py_v, pz_v, relbuf, deg1, deg2,
                  degtab_sp):
  s = lax.axis_index("s")
  c = lax.axis_index("c")
  wid = s * NC + c
  lane = jnp.arange(LANES, dtype=jnp.int32)

  pltpu.sync_copy(px_hbm, px_v)
  pltpu.sync_copy(py_hbm, py_v)
  pltpu.sync_copy(pz_hbm, pz_v)

  _zero_flat(relbuf, 4 * CH)
  _zero_2d4(deg1, CH)
  _zero_2d4(deg2, CH)
  _coop_zero(degtab_sp, deg1, s)
  plsc.subcore_barrier()
  ones = jnp.full((LANES,), 1.0, jnp.float32)
  for g in range(CH // LANES):
    rows = g * LANES + lane
    plsc.store_scatter(deg1, [rows, jnp.zeros((LANES,), jnp.int32)], ones)
    plsc.store_scatter(deg2, [rows, jnp.ones((LANES,), jnp.int32)], ones)

  for (sh, dh, e_pad, rel_out, degbuf) in (
      (si_hbm, di_hbm, e_i, reli_out, deg1),
      (ss_hbm, ds_hbm, e_s, rels_out, deg2)):
    ew = e_pad // NW
    def chunk(i, _):
      base = wid * ew + i * CH
      pltpu.sync_copy(sh.at[pl.ds(base, CH)], src_v)
      pltpu.sync_copy(dh.at[pl.ds(base, CH)], dst_v)
      def group(g, _):
        si = src_v[pl.ds(g * LANES, LANES)]
        di = dst_v[pl.ds(g * LANES, LANES)]
        flat0 = (g * LANES + lane) * 4
        for comp, tabv in enumerate((px_v, py_v, pz_v)):
          sv = plsc.load_gather(tabv, [si])
          dv = plsc.load_gather(tabv, [di])
          plsc.store_scatter(relbuf, [flat0 + comp], sv - dv)
        return 0
      lax.fori_loop(0, CH // LANES, group, 0)
      pltpu.sync_copy(relbuf, rel_out.at[pl.ds(base * 4, CH * 4)])
      pltpu.sync_copy(degbuf, degtab_sp.at[dst_v], add=True)
      return 0
    lax.fori_loop(0, ew // CH, chunk, 0)

  plsc.subcore_barrier()
  @pl.when(s == 0)
  def _():
    pltpu.sync_copy(degtab_sp.at[pl.ds(0, N_NODES)],
                    deg_out.at[pl.ds(c * N_NODES, N_NODES)])


def _sc_prep(si_i, di_i, ss_s, ds_s, px, py, pz, e_i, e_s):
  kfn = pl.kernel(
      functools.partial(_sc_prep_body, e_i, e_s),
      out_type=[
          jax.ShapeDtypeStruct((e_i * 4,), jnp.float32),
          jax.ShapeDtypeStruct((e_s * 4,), jnp.float32),
          jax.ShapeDtypeStruct((NC * N_NODES, 4), jnp.float32),
      ],
      mesh=_sc_mesh(),
      compiler_params=_SC_PARAMS,
      scratch_types=[
          pltpu.VMEM((CH,), jnp.int32),
          pltpu.VMEM((CH,), jnp.int32),
          pltpu.VMEM((NROWS,), jnp.float32),
          pltpu.VMEM((NROWS,), jnp.float32),
          pltpu.VMEM((NROWS,), jnp.float32),
          pltpu.VMEM((4 * CH,), jnp.float32),
          pltpu.VMEM((CH, 4), jnp.float32),
          pltpu.VMEM((CH, 4), jnp.float32),
          pltpu.VMEM_SHARED((NROWS, 4), jnp.float32),
      ],
  )
  return kfn(si_i, di_i, ss_s, ds_s, px, py, pz)


def _sc_edge_body(has_ea, e_pad, hw_hbm, ea_hbm, src_hbm, dst_hbm,
                  rel_hbm, gv_hbm,
                  agg_out, vd_out,
                  src_v0, src_v1, dst_v0, dst_v1, rows_v0, rows_v1,
                  ea_v, relv, gl_v, stage0, stage1,
                  agg_sp, vtab_sp,
                  sem_s0, sem_s1, sem_g0, sem_g1, sem_m, sem_c0, sem_c1):
  s = lax.axis_index("s")
  c = lax.axis_index("c")
  wid = s * NC + c
  ew = e_pad // NW
  nch = ew // CH
  lane = jnp.arange(LANES, dtype=jnp.int32)
  lane4 = lane // 4
  lmod4 = lane % 4
  bfly = [jnp.arange(LANES, dtype=jnp.int32) ^ sh for sh in (8, 4, 2, 1)]
  src_v = (src_v0, src_v1)
  dst_v = (dst_v0, dst_v1)
  rows_v = (rows_v0, rows_v1)
  stage = (stage0, stage1)
  sem_s = (sem_s0, sem_s1)
  sem_g = (sem_g0, sem_g1)
  sem_c = (sem_c0, sem_c1)

  pltpu.sync_copy(gv_hbm, gl_v)
  gl_regs = [gl_v[pl.ds(j * LANES, LANES)] for j in range(H // LANES)]

  def zrow(i, _):
    for j in range(H // LANES):
      rows_v0[i, pl.ds(j * LANES, LANES)] = jnp.zeros((LANES,), jnp.float32)
    return 0
  lax.fori_loop(0, CH, zrow, 0)
  _zero_2d4(stage0, CH)
  _coop_zero(agg_sp, rows_v0, s)
  _coop_zero(vtab_sp, stage0, s)
  plsc.subcore_barrier()

  def srcidx_cp(i, b):
    base = wid * ew + i * CH
    return pltpu.make_async_copy(src_hbm.at[pl.ds(base, CH)], src_v[b],
                                 sem_s[b])

  def gather_cp(i, b):
    return pltpu.make_async_copy(hw_hbm.at[src_v[b]], rows_v[b], sem_g[b])

  def misc_cps(i, b):
    base = wid * ew + i * CH
    cps = [pltpu.make_async_copy(dst_hbm.at[pl.ds(base, CH)], dst_v[b],
                                 sem_m),
           pltpu.make_async_copy(rel_hbm.at[pl.ds(base * 4, CH * 4)],
                                 relv, sem_m)]
    if has_ea:
      cps.append(pltpu.make_async_copy(ea_hbm.at[pl.ds(base, CH)], ea_v,
                                       sem_m))
    return cps

  def scat_cps(b):
    return (pltpu.make_async_copy(rows_v[b], agg_sp.at[dst_v[b]], sem_c[b]),
            pltpu.make_async_copy(stage[b], vtab_sp.at[dst_v[b]], sem_c[b]))

  def compute(i, b):
    rv = rows_v[b]
    st = stage[b]

    def group(g, _):
      dots = jnp.zeros((LANES,), jnp.float32)
      for k in range(LANES):
        e = g * LANES + k
        acc = jnp.zeros((LANES,), jnp.float32)
        for j in range(H // LANES):
          r = rv[e, pl.ds(j * LANES, LANES)]
          if has_ea:
            r = r + ea_v[e, pl.ds(j * LANES, LANES)]
          m = r / (1.0 + jnp.exp(-r))
          rv[e, pl.ds(j * LANES, LANES)] = m
          acc = acc + m * gl_regs[j]
        for perm in bfly:
          acc = acc + acc.at[perm].get(mode='promise_in_bounds')
        dots = jnp.where(lane == k, acc, dots)
      a = jnp.abs(dots)
      t = 1.0 - 2.0 / (jnp.exp(2.0 * a) + 1.0)
      gate = jnp.where(dots < 0.0, -t, t)
      for q in range(4):
        e0 = g * LANES + q * 4
        rvv = relv[pl.ds(e0 * 4, LANES)]
        g4 = gate.at[q * 4 + lane4].get(mode='promise_in_bounds')
        plsc.store_scatter(st, [e0 + lane4, lmod4], rvv * g4)
      return 0

    lax.fori_loop(0, CH // LANES, group, 0)

  srcidx_cp(0, 0).start()
  srcidx_cp(0, 0).wait()
  gather_cp(0, 0).start()
  if nch > 1:
    srcidx_cp(1, 1).start()
  for cp in misc_cps(0, 0):
    cp.start()
    cp.wait()

  def step(i, b):
    @pl.when(i + 1 < nch)
    def _():
      srcidx_cp(i + 1, 1 - b).wait()
    @pl.when(i > 0)
    def _():
      for cp in scat_cps(1 - b):
        cp.wait()
    gather_cp(i, b).wait()
    @pl.when(i + 1 < nch)
    def _():
      gather_cp(i + 1, 1 - b).start()
    @pl.when(i + 2 < nch)
    def _():
      srcidx_cp(i + 2, b).start()
    @pl.when(i > 0)
    def _():
      for cp in misc_cps(i, b):
        cp.wait()
    compute(i, b)
    for cp in scat_cps(b):
      cp.start(add=True)
    @pl.when(i + 1 < nch)
    def _():
      for cp in misc_cps(i + 1, 1 - b):
        cp.start()

  def pair(j, _):
    step(2 * j, 0)
    step(2 * j + 1, 1)
    return 0
  lax.fori_loop(0, nch // 2, pair, 0)
  if nch % 2:
    step(nch - 1, 0)
  for cp in scat_cps((nch - 1) % 2):
    cp.wait()

  plsc.subcore_barrier()

  @pl.when(s == 0)
  def _():
    pltpu.sync_copy(agg_sp.at[pl.ds(0, N_NODES)],
                    agg_out.at[pl.ds(c * N_NODES, N_NODES)])
    pltpu.sync_copy(vtab_sp.at[pl.ds(0, N_NODES)],
                    vd_out.at[pl.ds(c * N_NODES, N_NODES)])


def _sc_edge_phase(hw, ea, src_pad, dst_pad, rel, gv, e_pad, has_ea):
  ea_shape = (CH, H) if has_ea else (LANES,)
  scratch = [
      pltpu.VMEM((CH,), jnp.int32),
      pltpu.VMEM((CH,), jnp.int32),
      pltpu.VMEM((CH,), jnp.int32),
      pltpu.VMEM((CH,), jnp.int32),
      pltpu.VMEM((CH, H), jnp.float32),
      pltpu.VMEM((CH, H), jnp.float32),
      pltpu.VMEM(ea_shape, jnp.float32),
      pltpu.VMEM((4 * CH,), jnp.float32),
      pltpu.VMEM((H,), jnp.float32),
      pltpu.VMEM((CH, 4), jnp.float32),
      pltpu.VMEM((CH, 4), jnp.float32),
      pltpu.VMEM_SHARED((NROWS, H), jnp.float32),
      pltpu.VMEM_SHARED((NROWS, 4), jnp.float32),
      pltpu.SemaphoreType.DMA,
      pltpu.SemaphoreType.DMA,
      pltpu.SemaphoreType.DMA,
      pltpu.SemaphoreType.DMA,
      pltpu.SemaphoreType.DMA,
      pltpu.SemaphoreType.DMA,
      pltpu.SemaphoreType.DMA,
  ]
  out_type = [
      jax.ShapeDtypeStruct((NC * N_NODES, H), jnp.float32),
      jax.ShapeDtypeStruct((NC * N_NODES, 4), jnp.float32),
  ]
  kfn = pl.kernel(
      functools.partial(_sc_edge_body, has_ea, e_pad),
      out_type=out_type,
      mesh=_sc_mesh(),
      compiler_params=_SC_PARAMS,
      scratch_types=scratch,
  )
  if not has_ea:
    ea = jnp.zeros((LANES,), jnp.float32)
  return kfn(hw, ea, src_pad, dst_pad, rel, gv)


def kernel(x, edge_index_intra, edge_index_inter, pos, edge_attr, batch,
           params):
  p = params
  e_i = _epad(edge_index_intra.shape[1])
  e_s = _epad(edge_index_inter.shape[1])
  si_i, di_i = _pad_edges(edge_index_intra[0], edge_index_intra[1], e_i)
  ss_s, ds_s = _pad_edges(edge_index_inter[0], edge_index_inter[1], e_s)
  ea_pad = jnp.concatenate(
      [edge_attr,
       jnp.zeros((e_i - edge_attr.shape[0], D_EDGE), jnp.float32)])
  posp = jnp.concatenate([pos, jnp.zeros((NROWS - N_NODES, 3), jnp.float32)])
  px = posp[:, 0].copy()
  py = posp[:, 1].copy()
  pz = posp[:, 2].copy()

  we_all = jnp.stack([p['We%d' % l] for l in range(4)])
  bi_all = jnp.stack([p['bi%d' % l].reshape(1, H) for l in range(4)])
  eab = _tc_edgefeat(ea_pad, we_all, bi_all, e_i)

  h = _tc_embed(x, p['Wn'], p['bn'])

  rel_i, rel_s, deg_p = _sc_prep(si_i, di_i, ss_s, ds_s, px, py, pz, e_i, e_s)
  recip, scale = _tc_degfin(deg_p.reshape(NC, N_NODES, 4))

  vl = jnp.zeros((N_NODES, 4), jnp.float32)
  vp = jnp.zeros((N_NODES, 4), jnp.float32)
  for l in range(4):
    hwi, hws = _tc_proj(h, p['Wi%d' % l], p['Ws%d' % l], p['bs%d' % l])
    aggi_p, vld_p = _sc_edge_phase(
        hwi, eab[l], si_i, di_i, rel_i, p['gl%d' % l], e_i, True)
    aggs_p, vpd_p = _sc_edge_phase(
        hws, None, ss_s, ds_s, rel_s, p['gp%d' % l], e_s, False)
    h, vl, vp = _tc_update(h, aggi_p.reshape(NC, N_NODES, H),
                           aggs_p.reshape(NC, N_NODES, H),
                           vld_p.reshape(NC, N_NODES, 4),
                           vpd_p.reshape(NC, N_NODES, 4), vl, vp,
                           recip, scale)

  fw_all = jnp.stack([p['fW%d' % j] for j in range(3)])
  fb_all = jnp.stack([p['fb%d' % j].reshape(1, H) for j in range(3)])
  gam_all = jnp.stack([p['gamma%d' % j].reshape(1, H) for j in range(3)])
  bet_all = jnp.stack([p['beta%d' % j].reshape(1, H) for j in range(3)])
  out = _tc_head(h, batch.reshape(1, N_NODES), fw_all, fb_all, gam_all,
                 bet_all, p['fWout'], p['fbout'])
  return out.reshape(-1)


# speedup vs baseline: 1.3823x; 1.3100x over previous
import functools
import jax
import jax.numpy as jnp
from jax import lax
from jax.experimental import pallas as pl
from jax.experimental.pallas import tpu as pltpu
from jax.experimental.pallas import tpu_sc as plsc

N_NODES = 10000
D_NODE = 128
D_EDGE = 16
H = 128
NUM_GRAPHS = 64

NC = 2
NS = 16
NW = NC * NS
CH = 128
LANES = 16

NROWS = 10048
DUMMY = N_NODES

_SC_PARAMS = pltpu.CompilerParams(
    needs_layout_passes=False, use_tc_tiling_on_sc=False)


def _sc_mesh():
  return plsc.VectorSubcoreMesh(core_axis_name="c", subcore_axis_name="s",
                                num_cores=NC, num_subcores=NS)


def _pad_edges(e_src, e_dst, n_pad):
  pe = n_pad - e_src.shape[0]
  src = jnp.concatenate([e_src, jnp.zeros((pe,), jnp.int32)])
  dst = jnp.concatenate([e_dst, jnp.full((pe,), DUMMY, jnp.int32)])
  return src, dst


def _epad(e):
  per = NW * CH
  return ((e + per - 1) // per) * per


def _embed_body(x_ref, wn_ref, bn_ref, out_ref):
  z = jnp.dot(x_ref[...], wn_ref[...], preferred_element_type=jnp.float32)
  z = z + bn_ref[...]
  out_ref[...] = z / (1.0 + jnp.exp(-z))


def _tc_embed(x, wn, bn):
  return pl.pallas_call(
      _embed_body,
      out_shape=jax.ShapeDtypeStruct((N_NODES, H), jnp.float32),
  )(x, wn, bn.reshape(1, H))


def _edgefeat_body(ea_ref, we_ref, bi_ref, o0, o1, o2, o3):
  ea = ea_ref[...]
  outs = (o0, o1, o2, o3)
  for l in range(4):
    z = jnp.dot(ea, we_ref[l], preferred_element_type=jnp.float32)
    outs[l][...] = z + bi_ref[l]


def _tc_edgefeat(ea_pad, we_all, bi_all, e_pad):
  blk = NW * CH
  grid = e_pad // blk
  outs = [jax.ShapeDtypeStruct((e_pad, H), jnp.float32)] * 4
  return pl.pallas_call(
      _edgefeat_body,
      grid=(grid,),
      in_specs=[
          pl.BlockSpec((blk, D_EDGE), lambda i: (i, 0)),
          pl.BlockSpec((4, D_EDGE, H), lambda i: (0, 0, 0)),
          pl.BlockSpec((4, 1, H), lambda i: (0, 0, 0)),
      ],
      out_specs=[pl.BlockSpec((blk, H), lambda i: (i, 0))] * 4,
      out_shape=outs,
  )(ea_pad, we_all, bi_all)


def _proj_body(h_ref, wi_ref, ws_ref, bs_ref, oi_ref, os_ref):
  h = h_ref[...]
  oi_ref[...] = jnp.dot(h, wi_ref[...], preferred_element_type=jnp.float32)
  os_ref[...] = (jnp.dot(h, ws_ref[...], preferred_element_type=jnp.float32)
                 + bs_ref[...])


def _tc_proj(h, wi, ws, bs):
  return pl.pallas_call(
      _proj_body,
      out_shape=[jax.ShapeDtypeStruct((N_NODES, H), jnp.float32)] * 2,
  )(h, wi, ws, bs.reshape(1, H))


def _degfin_body(dp_ref, recip_ref, scale_ref):
  d = dp_ref[0] + dp_ref[1]
  recip_ref[...] = 1.0 / (d[:, 0:1] + 1.0)
  scale_ref[...] = jnp.log(d[:, 1:2] + 1.0)


def _tc_degfin(deg_p):
  return pl.pallas_call(
      _degfin_body,
      out_shape=[jax.ShapeDtypeStruct((N_NODES, 1), jnp.float32)] * 2,
  )(deg_p)


def _update_body(h_ref, ai_ref, as_ref, vld_ref, vpd_ref, vl_ref, vp_ref,
                 recip_ref, scale_ref, ho_ref, vlo_ref, vpo_ref):
  vl = vl_ref[...] + vld_ref[0] + vld_ref[1]
  vp = vp_ref[...] + vpd_ref[0] + vpd_ref[1]
  vlo_ref[...] = vl
  vpo_ref[...] = vp
  coup = jnp.tanh(jnp.sum(vl * vp, axis=1, keepdims=True))
  aggi = (ai_ref[0] + ai_ref[1]) * recip_ref[...]
  aggs = (as_ref[0] + as_ref[1]) * scale_ref[...]
  ho_ref[...] = h_ref[...] + aggi + aggs + 0.1 * coup


def _tc_update(h, aggi_p, aggs_p, vld_p, vpd_p, vl, vp, recip, scale):
  rb = 2000
  grid = N_NODES // rb
  return pl.pallas_call(
      _update_body,
      grid=(grid,),
      in_specs=[
          pl.BlockSpec((rb, H), lambda i: (i, 0)),
          pl.BlockSpec((NC, rb, H), lambda i: (0, i, 0)),
          pl.BlockSpec((NC, rb, H), lambda i: (0, i, 0)),
          pl.BlockSpec((NC, rb, 4), lambda i: (0, i, 0)),
          pl.BlockSpec((NC, rb, 4), lambda i: (0, i, 0)),
          pl.BlockSpec((rb, 4), lambda i: (i, 0)),
          pl.BlockSpec((rb, 4), lambda i: (i, 0)),
          pl.BlockSpec((rb, 1), lambda i: (i, 0)),
          pl.BlockSpec((rb, 1), lambda i: (i, 0)),
      ],
      out_specs=[
          pl.BlockSpec((rb, H), lambda i: (i, 0)),
          pl.BlockSpec((rb, 4), lambda i: (i, 0)),
          pl.BlockSpec((rb, 4), lambda i: (i, 0)),
      ],
      out_shape=[
          jax.ShapeDtypeStruct((N_NODES, H), jnp.float32),
          jax.ShapeDtypeStruct((N_NODES, 4), jnp.float32),
          jax.ShapeDtypeStruct((N_NODES, 4), jnp.float32),
      ],
  )(h, aggi_p, aggs_p, vld_p, vpd_p, vl, vp, recip, scale)


def _head_body(h_ref, b_ref, fw_ref, fb_ref, gam_ref, bet_ref,
               fwo_ref, fbo_ref, out_ref):
  gid = lax.broadcasted_iota(jnp.int32, (NUM_GRAPHS, 1), 0)
  onehot = (gid == b_ref[...]).astype(jnp.float32)
  g = jnp.dot(onehot, h_ref[...], preferred_element_type=jnp.float32)
  for j in range(3):
    g = jnp.dot(g, fw_ref[j], preferred_element_type=jnp.float32) + fb_ref[j]
    g = jnp.where(g > 0, g, 0.01 * g)
    mu = jnp.mean(g, axis=0)
    d = g - mu
    var = jnp.mean(d * d, axis=0)
    g = gam_ref[j] * d / jnp.sqrt(var + 1e-5) + bet_ref[j]
  out_ref[...] = (jnp.dot(g, fwo_ref[...], preferred_element_type=jnp.float32)
                  + fbo_ref[...])


def _tc_head(h, batch_row, fw_all, fb_all, gam_all, bet_all, fwo, fbo):
  return pl.pallas_call(
      _head_body,
      out_shape=jax.ShapeDtypeStruct((NUM_GRAPHS, 1), jnp.float32),
  )(h, batch_row, fw_all, fb_all, gam_all, bet_all, fwo, fbo.reshape(1, 1))


def _zero_flat(tab, n):
  z = jnp.zeros((LANES,), jnp.float32)
  def body(i, _):
    tab[pl.ds(i * LANES, LANES)] = z
    return 0
  lax.fori_loop(0, n // LANES, body, 0)


def _zero_2d4(tab, nrow):
  z = jnp.zeros((LANES,), jnp.float32)
  rows0 = jnp.arange(LANES, dtype=jnp.int32) // 4
  cols = jnp.arange(LANES, dtype=jnp.int32) % 4
  def body(i, _):
    plsc.store_scatter(tab, [i * 4 + rows0, cols], z)
    return 0
  lax.fori_loop(0, nrow // 4, body, 0)


def _coop_zero(sp_tab, zbuf, s):
  nblk = NROWS // 64
  def body(t, _):
    cid = t * NS + s
    @pl.when(cid < nblk)
    def _():
      pltpu.sync_copy(zbuf.at[pl.ds(0, 64)], sp_tab.at[pl.ds(cid * 64, 64)])
    return 0
  lax.fori_loop(0, (nblk + NS - 1) // NS, body, 0)


def _sc_prep_body(e_i, e_s, si_hbm, di_hbm, ss_hbm, ds_hbm,
                  px_hbm, py_hbm, pz_hbm,
                  reli_out, rels_out, deg_out,
                  src_v, dst_v, px_v, py_v, pz_v, relbuf, deg1, deg2,
                  degtab_sp):
  s = lax.axis_index("s")
  c = lax.axis_index("c")
  wid = s * NC + c
  lane = jnp.arange(LANES, dtype=jnp.int32)

  pltpu.sync_copy(px_hbm, px_v)
  pltpu.sync_copy(py_hbm, py_v)
  pltpu.sync_copy(pz_hbm, pz_v)

  _zero_flat(relbuf, 4 * CH)
  _zero_2d4(deg1, CH)
  _zero_2d4(deg2, CH)
  _coop_zero(degtab_sp, deg1, s)
  plsc.subcore_barrier()
  ones = jnp.full((LANES,), 1.0, jnp.float32)
  for g in range(CH // LANES):
    rows = g * LANES + lane
    plsc.store_scatter(deg1, [rows, jnp.zeros((LANES,), jnp.int32)], ones)
    plsc.store_scatter(deg2, [rows, jnp.ones((LANES,), jnp.int32)], ones)

  for (sh, dh, e_pad, rel_out, degbuf) in (
      (si_hbm, di_hbm, e_i, reli_out, deg1),
      (ss_hbm, ds_hbm, e_s, rels_out, deg2)):
    ew = e_pad // NW
    def chunk(i, _):
      base = wid * ew + i * CH
      pltpu.sync_copy(sh.at[pl.ds(base, CH)], src_v)
      pltpu.sync_copy(dh.at[pl.ds(base, CH)], dst_v)
      def group(g, _):
        si = src_v[pl.ds(g * LANES, LANES)]
        di = dst_v[pl.ds(g * LANES, LANES)]
        flat0 = (g * LANES + lane) * 4
        for comp, tabv in enumerate((px_v, py_v, pz_v)):
          sv = plsc.load_gather(tabv, [si])
          dv = plsc.load_gather(tabv, [di])
          plsc.store_scatter(relbuf, [flat0 + comp], sv - dv)
        return 0
      lax.fori_loop(0, CH // LANES, group, 0)
      pltpu.sync_copy(relbuf, rel_out.at[pl.ds(base * 4, CH * 4)])
      pltpu.sync_copy(degbuf, degtab_sp.at[dst_v], add=True)
      return 0
    lax.fori_loop(0, ew // CH, chunk, 0)

  plsc.subcore_barrier()
  @pl.when(s == 0)
  def _():
    pltpu.sync_copy(degtab_sp.at[pl.ds(0, N_NODES)],
                    deg_out.at[pl.ds(c * N_NODES, N_NODES)])


def _sc_prep(si_i, di_i, ss_s, ds_s, px, py, pz, e_i, e_s):
  kfn = pl.kernel(
      functools.partial(_sc_prep_body, e_i, e_s),
      out_type=[
          jax.ShapeDtypeStruct((e_i * 4,), jnp.float32),
          jax.ShapeDtypeStruct((e_s * 4,), jnp.float32),
          jax.ShapeDtypeStruct((NC * N_NODES, 4), jnp.float32),
      ],
      mesh=_sc_mesh(),
      compiler_params=_SC_PARAMS,
      scratch_types=[
          pltpu.VMEM((CH,), jnp.int32),
          pltpu.VMEM((CH,), jnp.int32),
          pltpu.VMEM((NROWS,), jnp.float32),
          pltpu.VMEM((NROWS,), jnp.float32),
          pltpu.VMEM((NROWS,), jnp.float32),
          pltpu.VMEM((4 * CH,), jnp.float32),
          pltpu.VMEM((CH, 4), jnp.float32),
          pltpu.VMEM((CH, 4), jnp.float32),
          pltpu.VMEM_SHARED((NROWS, 4), jnp.float32),
      ],
  )
  return kfn(si_i, di_i, ss_s, ds_s, px, py, pz)


def _sc_edge_body(has_ea, e_pad, hw_hbm, ea_hbm, src_hbm, dst_hbm,
                  rel_hbm, gv_hbm,
                  agg_out, vd_out,
                  src_v, dst_v, rows_v, ea_v, gl_v, relv, stage,
                  agg_sp, vtab_sp, gsem):
  s = lax.axis_index("s")
  c = lax.axis_index("c")
  wid = s * NC + c
  ew = e_pad // NW
  lane = jnp.arange(LANES, dtype=jnp.int32)
  lane4 = lane // 4
  lmod4 = lane % 4
  bfly = [jnp.arange(LANES, dtype=jnp.int32) ^ sh for sh in (8, 4, 2, 1)]

  pltpu.sync_copy(gv_hbm, gl_v)
  gl_regs = [gl_v[pl.ds(j * LANES, LANES)] for j in range(H // LANES)]

  def zrow(i, _):
    for j in range(H // LANES):
      rows_v[i, pl.ds(j * LANES, LANES)] = jnp.zeros((LANES,), jnp.float32)
    return 0
  lax.fori_loop(0, CH, zrow, 0)
  _zero_2d4(stage, CH)
  _coop_zero(agg_sp, rows_v, s)
  _coop_zero(vtab_sp, stage, s)
  plsc.subcore_barrier()

  def chunk(i, _):
    base = wid * ew + i * CH
    pltpu.sync_copy(src_hbm.at[pl.ds(base, CH)], src_v)
    pltpu.sync_copy(dst_hbm.at[pl.ds(base, CH)], dst_v)
    pltpu.async_copy(hw_hbm.at[src_v], rows_v, gsem).wait()
    if has_ea:
      pltpu.sync_copy(ea_hbm.at[pl.ds(base, CH)], ea_v)
    pltpu.sync_copy(rel_hbm.at[pl.ds(base * 4, CH * 4)], relv)

    def group(g, _):
      dots = jnp.zeros((LANES,), jnp.float32)
      for k in range(LANES):
        e = g * LANES + k
        acc = jnp.zeros((LANES,), jnp.float32)
        for j in range(H // LANES):
          r = rows_v[e, pl.ds(j * LANES, LANES)]
          if has_ea:
            r = r + ea_v[e, pl.ds(j * LANES, LANES)]
          m = r / (1.0 + jnp.exp(-r))
          rows_v[e, pl.ds(j * LANES, LANES)] = m
          acc = acc + m * gl_regs[j]
        for perm in bfly:
          acc = acc + acc.at[perm].get(mode='promise_in_bounds')
        dots = jnp.where(lane == k, acc, dots)
      a = jnp.abs(dots)
      t = 1.0 - 2.0 / (jnp.exp(2.0 * a) + 1.0)
      gate = jnp.where(dots < 0.0, -t, t)
      for q in range(4):
        e0 = g * LANES + q * 4
        rvv = relv[pl.ds(e0 * 4, LANES)]
        g4 = gate.at[q * 4 + lane4].get(mode='promise_in_bounds')
        plsc.store_scatter(stage, [e0 + lane4, lmod4], rvv * g4)
      return 0

    lax.fori_loop(0, CH // LANES, group, 0)
    pltpu.sync_copy(rows_v, agg_sp.at[dst_v], add=True)
    pltpu.sync_copy(stage, vtab_sp.at[dst_v], add=True)
    return 0

  lax.fori_loop(0, ew // CH, chunk, 0)
  plsc.subcore_barrier()

  @pl.when(s == 0)
  def _():
    pltpu.sync_copy(agg_sp.at[pl.ds(0, N_NODES)],
                    agg_out.at[pl.ds(c * N_NODES, N_NODES)])
    pltpu.sync_copy(vtab_sp.at[pl.ds(0, N_NODES)],
                    vd_out.at[pl.ds(c * N_NODES, N_NODES)])


def _sc_edge_phase(hw, ea, src_pad, dst_pad, rel, gv, e_pad, has_ea):
  scratch = [
      pltpu.VMEM((CH,), jnp.int32),
      pltpu.VMEM((CH,), jnp.int32),
      pltpu.VMEM((CH, H), jnp.float32),
      pltpu.VMEM((CH, H) if has_ea else (LANES,), jnp.float32),
      pltpu.VMEM((H,), jnp.float32),
      pltpu.VMEM((4 * CH,), jnp.float32),
      pltpu.VMEM((CH, 4), jnp.float32),
      pltpu.VMEM_SHARED((NROWS, H), jnp.float32),
      pltpu.VMEM_SHARED((NROWS, 4), jnp.float32),
      pltpu.SemaphoreType.DMA,
  ]
  out_type = [
      jax.ShapeDtypeStruct((NC * N_NODES, H), jnp.float32),
      jax.ShapeDtypeStruct((NC * N_NODES, 4), jnp.float32),
  ]
  kfn = pl.kernel(
      functools.partial(_sc_edge_body, has_ea, e_pad),
      out_type=out_type,
      mesh=_sc_mesh(),
      compiler_params=_SC_PARAMS,
      scratch_types=scratch,
  )
  if not has_ea:
    ea = jnp.zeros((LANES,), jnp.float32)
  return kfn(hw, ea, src_pad, dst_pad, rel, gv)


def kernel(x, edge_index_intra, edge_index_inter, pos, edge_attr, batch,
           params):
  p = params
  e_i = _epad(edge_index_intra.shape[1])
  e_s = _epad(edge_index_inter.shape[1])
  si_i, di_i = _pad_edges(edge_index_intra[0], edge_index_intra[1], e_i)
  ss_s, ds_s = _pad_edges(edge_index_inter[0], edge_index_inter[1], e_s)
  ea_pad = jnp.concatenate(
      [edge_attr,
       jnp.zeros((e_i - edge_attr.shape[0], D_EDGE), jnp.float32)])
  posp = jnp.concatenate([pos, jnp.zeros((NROWS - N_NODES, 3), jnp.float32)])
  px = posp[:, 0].copy()
  py = posp[:, 1].copy()
  pz = posp[:, 2].copy()

  we_all = jnp.stack([p['We%d' % l] for l in range(4)])
  bi_all = jnp.stack([p['bi%d' % l].reshape(1, H) for l in range(4)])
  eab = _tc_edgefeat(ea_pad, we_all, bi_all, e_i)

  h = _tc_embed(x, p['Wn'], p['bn'])

  rel_i, rel_s, deg_p = _sc_prep(si_i, di_i, ss_s, ds_s, px, py, pz, e_i, e_s)
  recip, scale = _tc_degfin(deg_p.reshape(NC, N_NODES, 4))

  vl = jnp.zeros((N_NODES, 4), jnp.float32)
  vp = jnp.zeros((N_NODES, 4), jnp.float32)
  for l in range(4):
    hwi, hws = _tc_proj(h, p['Wi%d' % l], p['Ws%d' % l], p['bs%d' % l])
    aggi_p, vld_p = _sc_edge_phase(
        hwi, eab[l], si_i, di_i, rel_i, p['gl%d' % l], e_i, True)
    aggs_p, vpd_p = _sc_edge_phase(
        hws, None, ss_s, ds_s, rel_s, p['gp%d' % l], e_s, False)
    h, vl, vp = _tc_update(h, aggi_p.reshape(NC, N_NODES, H),
                           aggs_p.reshape(NC, N_NODES, H),
                           vld_p.reshape(NC, N_NODES, 4),
                           vpd_p.reshape(NC, N_NODES, 4), vl, vp,
                           recip, scale)

  fw_all = jnp.stack([p['fW%d' % j] for j in range(3)])
  fb_all = jnp.stack([p['fb%d' % j].reshape(1, H) for j in range(3)])
  gam_all = jnp.stack([p['gamma%d' % j].reshape(1, H) for j in range(3)])
  bet_all = jnp.stack([p['beta%d' % j].reshape(1, H) for j in range(3)])
  out = _tc_head(h, batch.reshape(1, N_NODES), fw_all, fb_all, gam_all,
                 bet_all, p['fWout'], p['fbout'])
  return out.reshape(-1)


# speedup vs baseline: 1.4707x; 1.0639x over previous
import functools
import jax
import jax.numpy as jnp
from jax import lax
from jax.experimental import pallas as pl
from jax.experimental.pallas import tpu as pltpu
from jax.experimental.pallas import tpu_sc as plsc

N_NODES = 10000
D_NODE = 128
D_EDGE = 16
H = 128
NUM_GRAPHS = 64

NC = 2
NS = 16
NW = NC * NS
CH = 160
LANES = 16

NROWS = 10048
DUMMY = N_NODES

_SC_PARAMS = pltpu.CompilerParams(
    needs_layout_passes=False, use_tc_tiling_on_sc=False)


def _sc_mesh():
  return plsc.VectorSubcoreMesh(core_axis_name="c", subcore_axis_name="s",
                                num_cores=NC, num_subcores=NS)


def _pad_edges(e_src, e_dst, n_pad):
  pe = n_pad - e_src.shape[0]
  src = jnp.concatenate([e_src, jnp.zeros((pe,), jnp.int32)])
  dst = jnp.concatenate([e_dst, jnp.full((pe,), DUMMY, jnp.int32)])
  return src, dst


def _epad(e):
  per = NW * CH
  return ((e + per - 1) // per) * per


def _embed_body(x_ref, wn_ref, bn_ref, out_ref):
  z = jnp.dot(x_ref[...], wn_ref[...], preferred_element_type=jnp.float32)
  z = z + bn_ref[...]
  out_ref[...] = z / (1.0 + jnp.exp(-z))


def _tc_embed(x, wn, bn):
  return pl.pallas_call(
      _embed_body,
      out_shape=jax.ShapeDtypeStruct((N_NODES, H), jnp.float32),
  )(x, wn, bn.reshape(1, H))


def _edgefeat_body(ea_ref, we_ref, bi_ref, o0, o1, o2, o3):
  ea = ea_ref[...]
  outs = (o0, o1, o2, o3)
  for l in range(4):
    z = jnp.dot(ea, we_ref[l], preferred_element_type=jnp.float32)
    outs[l][...] = z + bi_ref[l]


def _tc_edgefeat(ea_pad, we_all, bi_all, e_pad):
  blk = NW * CH
  grid = e_pad // blk
  outs = [jax.ShapeDtypeStruct((e_pad, H), jnp.float32)] * 4
  return pl.pallas_call(
      _edgefeat_body,
      grid=(grid,),
      in_specs=[
          pl.BlockSpec((blk, D_EDGE), lambda i: (i, 0)),
          pl.BlockSpec((4, D_EDGE, H), lambda i: (0, 0, 0)),
          pl.BlockSpec((4, 1, H), lambda i: (0, 0, 0)),
      ],
      out_specs=[pl.BlockSpec((blk, H), lambda i: (i, 0))] * 4,
      out_shape=outs,
  )(ea_pad, we_all, bi_all)


def _proj_body(h_ref, wi_ref, ws_ref, bs_ref, oi_ref, os_ref):
  h = h_ref[...]
  oi_ref[...] = jnp.dot(h, wi_ref[...], preferred_element_type=jnp.float32)
  os_ref[...] = (jnp.dot(h, ws_ref[...], preferred_element_type=jnp.float32)
                 + bs_ref[...])


def _tc_proj(h, wi, ws, bs):
  return pl.pallas_call(
      _proj_body,
      out_shape=[jax.ShapeDtypeStruct((N_NODES, H), jnp.float32)] * 2,
  )(h, wi, ws, bs.reshape(1, H))


def _degfin_body(dp_ref, recip_ref, scale_ref):
  d = dp_ref[0] + dp_ref[1]
  recip_ref[...] = 1.0 / (d[:, 0:1] + 1.0)
  scale_ref[...] = jnp.log(d[:, 1:2] + 1.0)


def _tc_degfin(deg_p):
  return pl.pallas_call(
      _degfin_body,
      out_shape=[jax.ShapeDtypeStruct((N_NODES, 1), jnp.float32)] * 2,
  )(deg_p)


def _update_body(h_ref, ai_ref, as_ref, vld_ref, vpd_ref, vl_ref, vp_ref,
                 recip_ref, scale_ref, ho_ref, vlo_ref, vpo_ref):
  vl = vl_ref[...] + vld_ref[0] + vld_ref[1]
  vp = vp_ref[...] + vpd_ref[0] + vpd_ref[1]
  vlo_ref[...] = vl
  vpo_ref[...] = vp
  coup = jnp.tanh(jnp.sum(vl * vp, axis=1, keepdims=True))
  aggi = (ai_ref[0] + ai_ref[1]) * recip_ref[...]
  aggs = (as_ref[0] + as_ref[1]) * scale_ref[...]
  ho_ref[...] = h_ref[...] + aggi + aggs + 0.1 * coup


def _tc_update(h, aggi_p, aggs_p, vld_p, vpd_p, vl, vp, recip, scale):
  rb = 2000
  grid = N_NODES // rb
  return pl.pallas_call(
      _update_body,
      grid=(grid,),
      in_specs=[
          pl.BlockSpec((rb, H), lambda i: (i, 0)),
          pl.BlockSpec((NC, rb, H), lambda i: (0, i, 0)),
          pl.BlockSpec((NC, rb, H), lambda i: (0, i, 0)),
          pl.BlockSpec((NC, rb, 4), lambda i: (0, i, 0)),
          pl.BlockSpec((NC, rb, 4), lambda i: (0, i, 0)),
          pl.BlockSpec((rb, 4), lambda i: (i, 0)),
          pl.BlockSpec((rb, 4), lambda i: (i, 0)),
          pl.BlockSpec((rb, 1), lambda i: (i, 0)),
          pl.BlockSpec((rb, 1), lambda i: (i, 0)),
      ],
      out_specs=[
          pl.BlockSpec((rb, H), lambda i: (i, 0)),
          pl.BlockSpec((rb, 4), lambda i: (i, 0)),
          pl.BlockSpec((rb, 4), lambda i: (i, 0)),
      ],
      out_shape=[
          jax.ShapeDtypeStruct((N_NODES, H), jnp.float32),
          jax.ShapeDtypeStruct((N_NODES, 4), jnp.float32),
          jax.ShapeDtypeStruct((N_NODES, 4), jnp.float32),
      ],
  )(h, aggi_p, aggs_p, vld_p, vpd_p, vl, vp, recip, scale)


def _head_body(h_ref, b_ref, fw_ref, fb_ref, gam_ref, bet_ref,
               fwo_ref, fbo_ref, out_ref):
  gid = lax.broadcasted_iota(jnp.int32, (NUM_GRAPHS, 1), 0)
  onehot = (gid == b_ref[...]).astype(jnp.float32)
  g = jnp.dot(onehot, h_ref[...], preferred_element_type=jnp.float32)
  for j in range(3):
    g = jnp.dot(g, fw_ref[j], preferred_element_type=jnp.float32) + fb_ref[j]
    g = jnp.where(g > 0, g, 0.01 * g)
    mu = jnp.mean(g, axis=0)
    d = g - mu
    var = jnp.mean(d * d, axis=0)
    g = gam_ref[j] * d / jnp.sqrt(var + 1e-5) + bet_ref[j]
  out_ref[...] = (jnp.dot(g, fwo_ref[...], preferred_element_type=jnp.float32)
                  + fbo_ref[...])


def _tc_head(h, batch_row, fw_all, fb_all, gam_all, bet_all, fwo, fbo):
  return pl.pallas_call(
      _head_body,
      out_shape=jax.ShapeDtypeStruct((NUM_GRAPHS, 1), jnp.float32),
  )(h, batch_row, fw_all, fb_all, gam_all, bet_all, fwo, fbo.reshape(1, 1))


def _zero_flat(tab, n):
  z = jnp.zeros((LANES,), jnp.float32)
  def body(i, _):
    tab[pl.ds(i * LANES, LANES)] = z
    return 0
  lax.fori_loop(0, n // LANES, body, 0)


def _zero_2d4(tab, nrow):
  z = jnp.zeros((LANES,), jnp.float32)
  rows0 = jnp.arange(LANES, dtype=jnp.int32) // 4
  cols = jnp.arange(LANES, dtype=jnp.int32) % 4
  def body(i, _):
    plsc.store_scatter(tab, [i * 4 + rows0, cols], z)
    return 0
  lax.fori_loop(0, nrow // 4, body, 0)


def _coop_zero(sp_tab, zbuf, s):
  nblk = NROWS // 64
  def body(t, _):
    cid = t * NS + s
    @pl.when(cid < nblk)
    def _():
      pltpu.sync_copy(zbuf.at[pl.ds(0, 64)], sp_tab.at[pl.ds(cid * 64, 64)])
    return 0
  lax.fori_loop(0, (nblk + NS - 1) // NS, body, 0)


def _sc_prep_body(e_i, e_s, si_hbm, di_hbm, ss_hbm, ds_hbm,
                  px_hbm, py_hbm, pz_hbm,
                  reli_out, rels_out, deg_out,
                  src_v, dst_v, px_v, py_v, pz_v, relbuf, deg1, deg2,
                  degtab_sp):
  s = lax.axis_index("s")
  c = lax.axis_index("c")
  wid = s * NC + c
  lane = jnp.arange(LANES, dtype=jnp.int32)

  pltpu.sync_copy(px_hbm, px_v)
  pltpu.sync_copy(py_hbm, py_v)
  pltpu.sync_copy(pz_hbm, pz_v)

  _zero_flat(relbuf, 4 * CH)
  _zero_2d4(deg1, CH)
  _zero_2d4(deg2, CH)
  _coop_zero(degtab_sp, deg1, s)
  plsc.subcore_barrier()
  ones = jnp.full((LANES,), 1.0, jnp.float32)
  for g in range(CH // LANES):
    rows = g * LANES + lane
    plsc.store_scatter(deg1, [rows, jnp.zeros((LANES,), jnp.int32)], ones)
    plsc.store_scatter(deg2, [rows, jnp.ones((LANES,), jnp.int32)], ones)

  for (sh, dh, e_pad, rel_out, degbuf) in (
      (si_hbm, di_hbm, e_i, reli_out, deg1),
      (ss_hbm, ds_hbm, e_s, rels_out, deg2)):
    ew = e_pad // NW
    def chunk(i, _):
      base = wid * ew + i * CH
      pltpu.sync_copy(sh.at[pl.ds(base, CH)], src_v)
      pltpu.sync_copy(dh.at[pl.ds(base, CH)], dst_v)
      def group(g, _):
        si = src_v[pl.ds(g * LANES, LANES)]
        di = dst_v[pl.ds(g * LANES, LANES)]
        flat0 = (g * LANES + lane) * 4
        for comp, tabv in enumerate((px_v, py_v, pz_v)):
          sv = plsc.load_gather(tabv, [si])
          dv = plsc.load_gather(tabv, [di])
          plsc.store_scatter(relbuf, [flat0 + comp], sv - dv)
        return 0
      lax.fori_loop(0, CH // LANES, group, 0)
      pltpu.sync_copy(relbuf, rel_out.at[pl.ds(base * 4, CH * 4)])
      pltpu.sync_copy(degbuf, degtab_sp.at[dst_v], add=True)
      return 0
    lax.fori_loop(0, ew // CH, chunk, 0)

  plsc.subcore_barrier()
  @pl.when(s == 0)
  def _():
    pltpu.sync_copy(degtab_sp.at[pl.ds(0, N_NODES)],
                    deg_out.at[pl.ds(c * N_NODES, N_NODES)])


def _sc_prep(si_i, di_i, ss_s, ds_s, px, py, pz, e_i, e_s):
  kfn = pl.kernel(
      functools.partial(_sc_prep_body, e_i, e_s),
      out_type=[
          jax.ShapeDtypeStruct((e_i * 4,), jnp.float32),
          jax.ShapeDtypeStruct((e_s * 4,), jnp.float32),
          jax.ShapeDtypeStruct((NC * N_NODES, 4), jnp.float32),
      ],
      mesh=_sc_mesh(),
      compiler_params=_SC_PARAMS,
      scratch_types=[
          pltpu.VMEM((CH,), jnp.int32),
          pltpu.VMEM((CH,), jnp.int32),
          pltpu.VMEM((NROWS,), jnp.float32),
          pltpu.VMEM((NROWS,), jnp.float32),
          pltpu.VMEM((NROWS,), jnp.float32),
          pltpu.VMEM((4 * CH,), jnp.float32),
          pltpu.VMEM((CH, 4), jnp.float32),
          pltpu.VMEM((CH, 4), jnp.float32),
          pltpu.VMEM_SHARED((NROWS, 4), jnp.float32),
      ],
  )
  return kfn(si_i, di_i, ss_s, ds_s, px, py, pz)


def _sc_edge_body(has_ea, e_pad, hw_hbm, ea_hbm, src_hbm, dst_hbm,
                  rel_hbm, gv_hbm,
                  agg_out, vd_out,
                  src_v, dst_v, rows_v, ea_v, gl_v, relv, stage,
                  agg_sp, vtab_sp, gsem):
  s = lax.axis_index("s")
  c = lax.axis_index("c")
  wid = s * NC + c
  ew = e_pad // NW
  lane = jnp.arange(LANES, dtype=jnp.int32)
  lane4 = lane // 4
  lmod4 = lane % 4
  bfly = [jnp.arange(LANES, dtype=jnp.int32) ^ sh for sh in (8, 4, 2, 1)]

  pltpu.sync_copy(gv_hbm, gl_v)
  gl_regs = [gl_v[pl.ds(j * LANES, LANES)] for j in range(H // LANES)]

  def zrow(i, _):
    for j in range(H // LANES):
      rows_v[i, pl.ds(j * LANES, LANES)] = jnp.zeros((LANES,), jnp.float32)
    return 0
  lax.fori_loop(0, CH, zrow, 0)
  _zero_2d4(stage, CH)
  _coop_zero(agg_sp, rows_v, s)
  _coop_zero(vtab_sp, stage, s)
  plsc.subcore_barrier()

  def chunk(i, _):
    base = wid * ew + i * CH
    pltpu.sync_copy(src_hbm.at[pl.ds(base, CH)], src_v)
    pltpu.sync_copy(dst_hbm.at[pl.ds(base, CH)], dst_v)
    pltpu.async_copy(hw_hbm.at[src_v], rows_v, gsem).wait()
    if has_ea:
      pltpu.sync_copy(ea_hbm.at[pl.ds(base, CH)], ea_v)
    pltpu.sync_copy(rel_hbm.at[pl.ds(base * 4, CH * 4)], relv)

    def group(g, _):
      dots = jnp.zeros((LANES,), jnp.float32)
      for k in range(LANES):
        e = g * LANES + k
        acc = jnp.zeros((LANES,), jnp.float32)
        for j in range(H // LANES):
          r = rows_v[e, pl.ds(j * LANES, LANES)]
          if has_ea:
            r = r + ea_v[e, pl.ds(j * LANES, LANES)]
          m = r / (1.0 + jnp.exp(-r))
          rows_v[e, pl.ds(j * LANES, LANES)] = m
          acc = acc + m * gl_regs[j]
        for perm in bfly:
          acc = acc + acc.at[perm].get(mode='promise_in_bounds')
        dots = jnp.where(lane == k, acc, dots)
      a = jnp.abs(dots)
      t = 1.0 - 2.0 / (jnp.exp(2.0 * a) + 1.0)
      gate = jnp.where(dots < 0.0, -t, t)
      for q in range(4):
        e0 = g * LANES + q * 4
        rvv = relv[pl.ds(e0 * 4, LANES)]
        g4 = gate.at[q * 4 + lane4].get(mode='promise_in_bounds')
        plsc.store_scatter(stage, [e0 + lane4, lmod4], rvv * g4)
      return 0

    lax.fori_loop(0, CH // LANES, group, 0)
    pltpu.sync_copy(rows_v, agg_sp.at[dst_v], add=True)
    pltpu.sync_copy(stage, vtab_sp.at[dst_v], add=True)
    return 0

  lax.fori_loop(0, ew // CH, chunk, 0)
  plsc.subcore_barrier()

  @pl.when(s == 0)
  def _():
    pltpu.sync_copy(agg_sp.at[pl.ds(0, N_NODES)],
                    agg_out.at[pl.ds(c * N_NODES, N_NODES)])
    pltpu.sync_copy(vtab_sp.at[pl.ds(0, N_NODES)],
                    vd_out.at[pl.ds(c * N_NODES, N_NODES)])


def _sc_edge_phase(hw, ea, src_pad, dst_pad, rel, gv, e_pad, has_ea):
  scratch = [
      pltpu.VMEM((CH,), jnp.int32),
      pltpu.VMEM((CH,), jnp.int32),
      pltpu.VMEM((CH, H), jnp.float32),
      pltpu.VMEM((CH, H) if has_ea else (LANES,), jnp.float32),
      pltpu.VMEM((H,), jnp.float32),
      pltpu.VMEM((4 * CH,), jnp.float32),
      pltpu.VMEM((CH, 4), jnp.float32),
      pltpu.VMEM_SHARED((NROWS, H), jnp.float32),
      pltpu.VMEM_SHARED((NROWS, 4), jnp.float32),
      pltpu.SemaphoreType.DMA,
  ]
  out_type = [
      jax.ShapeDtypeStruct((NC * N_NODES, H), jnp.float32),
      jax.ShapeDtypeStruct((NC * N_NODES, 4), jnp.float32),
  ]
  kfn = pl.kernel(
      functools.partial(_sc_edge_body, has_ea, e_pad),
      out_type=out_type,
      mesh=_sc_mesh(),
      compiler_params=_SC_PARAMS,
      scratch_types=scratch,
  )
  if not has_ea:
    ea = jnp.zeros((LANES,), jnp.float32)
  return kfn(hw, ea, src_pad, dst_pad, rel, gv)


def kernel(x, edge_index_intra, edge_index_inter, pos, edge_attr, batch,
           params):
  p = params
  e_i = _epad(edge_index_intra.shape[1])
  e_s = _epad(edge_index_inter.shape[1])
  si_i, di_i = _pad_edges(edge_index_intra[0], edge_index_intra[1], e_i)
  ss_s, ds_s = _pad_edges(edge_index_inter[0], edge_index_inter[1], e_s)
  ea_pad = jnp.concatenate(
      [edge_attr,
       jnp.zeros((e_i - edge_attr.shape[0], D_EDGE), jnp.float32)])
  posp = jnp.concatenate([pos, jnp.zeros((NROWS - N_NODES, 3), jnp.float32)])
  px = posp[:, 0].copy()
  py = posp[:, 1].copy()
  pz = posp[:, 2].copy()

  we_all = jnp.stack([p['We%d' % l] for l in range(4)])
  bi_all = jnp.stack([p['bi%d' % l].reshape(1, H) for l in range(4)])
  eab = _tc_edgefeat(ea_pad, we_all, bi_all, e_i)

  h = _tc_embed(x, p['Wn'], p['bn'])

  rel_i, rel_s, deg_p = _sc_prep(si_i, di_i, ss_s, ds_s, px, py, pz, e_i, e_s)
  recip, scale = _tc_degfin(deg_p.reshape(NC, N_NODES, 4))

  vl = jnp.zeros((N_NODES, 4), jnp.float32)
  vp = jnp.zeros((N_NODES, 4), jnp.float32)
  for l in range(4):
    hwi, hws = _tc_proj(h, p['Wi%d' % l], p['Ws%d' % l], p['bs%d' % l])
    aggi_p, vld_p = _sc_edge_phase(
        hwi, eab[l], si_i, di_i, rel_i, p['gl%d' % l], e_i, True)
    aggs_p, vpd_p = _sc_edge_phase(
        hws, None, ss_s, ds_s, rel_s, p['gp%d' % l], e_s, False)
    h, vl, vp = _tc_update(h, aggi_p.reshape(NC, N_NODES, H),
                           aggs_p.reshape(NC, N_NODES, H),
                           vld_p.reshape(NC, N_NODES, 4),
                           vpd_p.reshape(NC, N_NODES, 4), vl, vp,
                           recip, scale)

  fw_all = jnp.stack([p['fW%d' % j] for j in range(3)])
  fb_all = jnp.stack([p['fb%d' % j].reshape(1, H) for j in range(3)])
  gam_all = jnp.stack([p['gamma%d' % j].reshape(1, H) for j in range(3)])
  bet_all = jnp.stack([p['beta%d' % j].reshape(1, H) for j in range(3)])
  out = _tc_head(h, batch.reshape(1, N_NODES), fw_all, fb_all, gam_all,
                 bet_all, p['fWout'], p['fbout'])
  return out.reshape(-1)


# speedup vs baseline: 1.4831x; 1.0084x over previous
import functools
import jax
import jax.numpy as jnp
from jax import lax
from jax.experimental import pallas as pl
from jax.experimental.pallas import tpu as pltpu
from jax.experimental.pallas import tpu_sc as plsc

N_NODES = 10000
D_NODE = 128
D_EDGE = 16
H = 128
NUM_GRAPHS = 64

NC = 2
NS = 16
NW = NC * NS
CH = 160
CH_S = 256
LANES = 16

NROWS = 10048
DUMMY = N_NODES

_SC_PARAMS = pltpu.CompilerParams(
    needs_layout_passes=False, use_tc_tiling_on_sc=False)


def _sc_mesh():
  return plsc.VectorSubcoreMesh(core_axis_name="c", subcore_axis_name="s",
                                num_cores=NC, num_subcores=NS)


def _pad_edges(e_src, e_dst, n_pad):
  pe = n_pad - e_src.shape[0]
  src = jnp.concatenate([e_src, jnp.zeros((pe,), jnp.int32)])
  dst = jnp.concatenate([e_dst, jnp.full((pe,), DUMMY, jnp.int32)])
  return src, dst


def _epad(e, ch):
  per = NW * ch
  return ((e + per - 1) // per) * per


def _embed_body(x_ref, wn_ref, bn_ref, out_ref):
  z = jnp.dot(x_ref[...], wn_ref[...], preferred_element_type=jnp.float32)
  z = z + bn_ref[...]
  out_ref[...] = z / (1.0 + jnp.exp(-z))


def _tc_embed(x, wn, bn):
  return pl.pallas_call(
      _embed_body,
      out_shape=jax.ShapeDtypeStruct((N_NODES, H), jnp.float32),
  )(x, wn, bn.reshape(1, H))


def _edgefeat_body(ea_ref, we_ref, bi_ref, o0, o1, o2, o3):
  ea = ea_ref[...]
  outs = (o0, o1, o2, o3)
  for l in range(4):
    z = jnp.dot(ea, we_ref[l], preferred_element_type=jnp.float32)
    outs[l][...] = z + bi_ref[l]


def _tc_edgefeat(ea_pad, we_all, bi_all, e_pad):
  blk = NW * CH
  grid = e_pad // blk
  outs = [jax.ShapeDtypeStruct((e_pad, H), jnp.float32)] * 4
  return pl.pallas_call(
      _edgefeat_body,
      grid=(grid,),
      in_specs=[
          pl.BlockSpec((blk, D_EDGE), lambda i: (i, 0)),
          pl.BlockSpec((4, D_EDGE, H), lambda i: (0, 0, 0)),
          pl.BlockSpec((4, 1, H), lambda i: (0, 0, 0)),
      ],
      out_specs=[pl.BlockSpec((blk, H), lambda i: (i, 0))] * 4,
      out_shape=outs,
  )(ea_pad, we_all, bi_all)


def _proj_body(h_ref, wi_ref, ws_ref, bs_ref, oi_ref, os_ref):
  h = h_ref[...]
  oi_ref[...] = jnp.dot(h, wi_ref[...], preferred_element_type=jnp.float32)
  os_ref[...] = (jnp.dot(h, ws_ref[...], preferred_element_type=jnp.float32)
                 + bs_ref[...])


def _tc_proj(h, wi, ws, bs):
  return pl.pallas_call(
      _proj_body,
      out_shape=[jax.ShapeDtypeStruct((N_NODES, H), jnp.float32)] * 2,
  )(h, wi, ws, bs.reshape(1, H))


def _degfin_body(dp_ref, recip_ref, scale_ref):
  d = dp_ref[0] + dp_ref[1]
  recip_ref[...] = 1.0 / (d[:, 0:1] + 1.0)
  scale_ref[...] = jnp.log(d[:, 1:2] + 1.0)


def _tc_degfin(deg_p):
  return pl.pallas_call(
      _degfin_body,
      out_shape=[jax.ShapeDtypeStruct((N_NODES, 1), jnp.float32)] * 2,
  )(deg_p)


def _update_body(h_ref, ai_ref, as_ref, vld_ref, vpd_ref, vl_ref, vp_ref,
                 recip_ref, scale_ref, ho_ref, vlo_ref, vpo_ref):
  vl = vl_ref[...] + vld_ref[0] + vld_ref[1]
  vp = vp_ref[...] + vpd_ref[0] + vpd_ref[1]
  vlo_ref[...] = vl
  vpo_ref[...] = vp
  coup = jnp.tanh(jnp.sum(vl * vp, axis=1, keepdims=True))
  aggi = (ai_ref[0] + ai_ref[1]) * recip_ref[...]
  aggs = (as_ref[0] + as_ref[1]) * scale_ref[...]
  ho_ref[...] = h_ref[...] + aggi + aggs + 0.1 * coup


def _tc_update(h, aggi_p, aggs_p, vld_p, vpd_p, vl, vp, recip, scale):
  rb = 2000
  grid = N_NODES // rb
  return pl.pallas_call(
      _update_body,
      grid=(grid,),
      in_specs=[
          pl.BlockSpec((rb, H), lambda i: (i, 0)),
          pl.BlockSpec((NC, rb, H), lambda i: (0, i, 0)),
          pl.BlockSpec((NC, rb, H), lambda i: (0, i, 0)),
          pl.BlockSpec((NC, rb, 4), lambda i: (0, i, 0)),
          pl.BlockSpec((NC, rb, 4), lambda i: (0, i, 0)),
          pl.BlockSpec((rb, 4), lambda i: (i, 0)),
          pl.BlockSpec((rb, 4), lambda i: (i, 0)),
          pl.BlockSpec((rb, 1), lambda i: (i, 0)),
          pl.BlockSpec((rb, 1), lambda i: (i, 0)),
      ],
      out_specs=[
          pl.BlockSpec((rb, H), lambda i: (i, 0)),
          pl.BlockSpec((rb, 4), lambda i: (i, 0)),
          pl.BlockSpec((rb, 4), lambda i: (i, 0)),
      ],
      out_shape=[
          jax.ShapeDtypeStruct((N_NODES, H), jnp.float32),
          jax.ShapeDtypeStruct((N_NODES, 4), jnp.float32),
          jax.ShapeDtypeStruct((N_NODES, 4), jnp.float32),
      ],
  )(h, aggi_p, aggs_p, vld_p, vpd_p, vl, vp, recip, scale)


def _head_body(h_ref, b_ref, fw_ref, fb_ref, gam_ref, bet_ref,
               fwo_ref, fbo_ref, out_ref):
  gid = lax.broadcasted_iota(jnp.int32, (NUM_GRAPHS, 1), 0)
  onehot = (gid == b_ref[...]).astype(jnp.float32)
  g = jnp.dot(onehot, h_ref[...], preferred_element_type=jnp.float32)
  for j in range(3):
    g = jnp.dot(g, fw_ref[j], preferred_element_type=jnp.float32) + fb_ref[j]
    g = jnp.where(g > 0, g, 0.01 * g)
    mu = jnp.mean(g, axis=0)
    d = g - mu
    var = jnp.mean(d * d, axis=0)
    g = gam_ref[j] * d / jnp.sqrt(var + 1e-5) + bet_ref[j]
  out_ref[...] = (jnp.dot(g, fwo_ref[...], preferred_element_type=jnp.float32)
                  + fbo_ref[...])


def _tc_head(h, batch_row, fw_all, fb_all, gam_all, bet_all, fwo, fbo):
  return pl.pallas_call(
      _head_body,
      out_shape=jax.ShapeDtypeStruct((NUM_GRAPHS, 1), jnp.float32),
  )(h, batch_row, fw_all, fb_all, gam_all, bet_all, fwo, fbo.reshape(1, 1))


def _zero_flat(tab, n):
  z = jnp.zeros((LANES,), jnp.float32)
  def body(i, _):
    tab[pl.ds(i * LANES, LANES)] = z
    return 0
  lax.fori_loop(0, n // LANES, body, 0)


def _zero_2d4(tab, nrow):
  z = jnp.zeros((LANES,), jnp.float32)
  rows0 = jnp.arange(LANES, dtype=jnp.int32) // 4
  cols = jnp.arange(LANES, dtype=jnp.int32) % 4
  def body(i, _):
    plsc.store_scatter(tab, [i * 4 + rows0, cols], z)
    return 0
  lax.fori_loop(0, nrow // 4, body, 0)


def _coop_zero(sp_tab, zbuf, s):
  nblk = NROWS // 64
  def body(t, _):
    cid = t * NS + s
    @pl.when(cid < nblk)
    def _():
      pltpu.sync_copy(zbuf.at[pl.ds(0, 64)], sp_tab.at[pl.ds(cid * 64, 64)])
    return 0
  lax.fori_loop(0, (nblk + NS - 1) // NS, body, 0)


def _sc_prep_body(e_i, e_s, si_hbm, di_hbm, ss_hbm, ds_hbm,
                  px_hbm, py_hbm, pz_hbm,
                  reli_out, rels_out, deg_out,
                  src_v, dst_v, px_v, py_v, pz_v, relbuf, deg1, deg2,
                  degtab_sp):
  s = lax.axis_index("s")
  c = lax.axis_index("c")
  wid = s * NC + c
  lane = jnp.arange(LANES, dtype=jnp.int32)

  pltpu.sync_copy(px_hbm, px_v)
  pltpu.sync_copy(py_hbm, py_v)
  pltpu.sync_copy(pz_hbm, pz_v)

  _zero_flat(relbuf, 4 * CH)
  _zero_2d4(deg1, CH)
  _zero_2d4(deg2, CH)
  _coop_zero(degtab_sp, deg1, s)
  plsc.subcore_barrier()
  ones = jnp.full((LANES,), 1.0, jnp.float32)
  for g in range(CH // LANES):
    rows = g * LANES + lane
    plsc.store_scatter(deg1, [rows, jnp.zeros((LANES,), jnp.int32)], ones)
    plsc.store_scatter(deg2, [rows, jnp.ones((LANES,), jnp.int32)], ones)

  for (sh, dh, e_pad, rel_out, degbuf) in (
      (si_hbm, di_hbm, e_i, reli_out, deg1),
      (ss_hbm, ds_hbm, e_s, rels_out, deg2)):
    ew = e_pad // NW
    def chunk(i, _):
      base = wid * ew + i * CH
      pltpu.sync_copy(sh.at[pl.ds(base, CH)], src_v)
      pltpu.sync_copy(dh.at[pl.ds(base, CH)], dst_v)
      def group(g, _):
        si = src_v[pl.ds(g * LANES, LANES)]
        di = dst_v[pl.ds(g * LANES, LANES)]
        flat0 = (g * LANES + lane) * 4
        for comp, tabv in enumerate((px_v, py_v, pz_v)):
          sv = plsc.load_gather(tabv, [si])
          dv = plsc.load_gather(tabv, [di])
          plsc.store_scatter(relbuf, [flat0 + comp], sv - dv)
        return 0
      lax.fori_loop(0, CH // LANES, group, 0)
      pltpu.sync_copy(relbuf, rel_out.at[pl.ds(base * 4, CH * 4)])
      pltpu.sync_copy(degbuf, degtab_sp.at[dst_v], add=True)
      return 0
    lax.fori_loop(0, ew // CH, chunk, 0)

  plsc.subcore_barrier()
  @pl.when(s == 0)
  def _():
    pltpu.sync_copy(degtab_sp.at[pl.ds(0, N_NODES)],
                    deg_out.at[pl.ds(c * N_NODES, N_NODES)])


def _sc_prep(si_i, di_i, ss_s, ds_s, px, py, pz, e_i, e_s):
  kfn = pl.kernel(
      functools.partial(_sc_prep_body, e_i, e_s),
      out_type=[
          jax.ShapeDtypeStruct((e_i * 4,), jnp.float32),
          jax.ShapeDtypeStruct((e_s * 4,), jnp.float32),
          jax.ShapeDtypeStruct((NC * N_NODES, 4), jnp.float32),
      ],
      mesh=_sc_mesh(),
      compiler_params=_SC_PARAMS,
      scratch_types=[
          pltpu.VMEM((CH,), jnp.int32),
          pltpu.VMEM((CH,), jnp.int32),
          pltpu.VMEM((NROWS,), jnp.float32),
          pltpu.VMEM((NROWS,), jnp.float32),
          pltpu.VMEM((NROWS,), jnp.float32),
          pltpu.VMEM((4 * CH,), jnp.float32),
          pltpu.VMEM((CH, 4), jnp.float32),
          pltpu.VMEM((CH, 4), jnp.float32),
          pltpu.VMEM_SHARED((NROWS, 4), jnp.float32),
      ],
  )
  return kfn(si_i, di_i, ss_s, ds_s, px, py, pz)


def _sc_edge_body(has_ea, e_pad, ch, hw_hbm, ea_hbm, src_hbm, dst_hbm,
                  rel_hbm, gv_hbm,
                  agg_out, vd_out,
                  src_v, dst_v, rows_v, ea_v, gl_v, relv, stage,
                  agg_sp, vtab_sp, gsem):
  s = lax.axis_index("s")
  c = lax.axis_index("c")
  wid = s * NC + c
  ew = e_pad // NW
  lane = jnp.arange(LANES, dtype=jnp.int32)
  lane4 = lane // 4
  lmod4 = lane % 4
  bfly = [jnp.arange(LANES, dtype=jnp.int32) ^ sh for sh in (8, 4, 2, 1)]

  pltpu.sync_copy(gv_hbm, gl_v)
  gl_regs = [gl_v[pl.ds(j * LANES, LANES)] for j in range(H // LANES)]

  def zrow(i, _):
    for j in range(H // LANES):
      rows_v[i, pl.ds(j * LANES, LANES)] = jnp.zeros((LANES,), jnp.float32)
    return 0
  lax.fori_loop(0, ch, zrow, 0)
  _zero_2d4(stage, ch)
  _coop_zero(agg_sp, rows_v, s)
  _coop_zero(vtab_sp, stage, s)
  plsc.subcore_barrier()

  def chunk(i, _):
    base = wid * ew + i * ch
    pltpu.sync_copy(src_hbm.at[pl.ds(base, ch)], src_v)
    pltpu.sync_copy(dst_hbm.at[pl.ds(base, ch)], dst_v)
    pltpu.async_copy(hw_hbm.at[src_v], rows_v, gsem).wait()
    if has_ea:
      pltpu.sync_copy(ea_hbm.at[pl.ds(base, ch)], ea_v)
    pltpu.sync_copy(rel_hbm.at[pl.ds(base * 4, ch * 4)], relv)

    def group(g, _):
      dots = jnp.zeros((LANES,), jnp.float32)
      for k in range(LANES):
        e = g * LANES + k
        acc = jnp.zeros((LANES,), jnp.float32)
        for j in range(H // LANES):
          r = rows_v[e, pl.ds(j * LANES, LANES)]
          if has_ea:
            r = r + ea_v[e, pl.ds(j * LANES, LANES)]
          m = r / (1.0 + jnp.exp(-r))
          rows_v[e, pl.ds(j * LANES, LANES)] = m
          acc = acc + m * gl_regs[j]
        for perm in bfly:
          acc = acc + acc.at[perm].get(mode='promise_in_bounds')
        dots = jnp.where(lane == k, acc, dots)
      a = jnp.abs(dots)
      t = 1.0 - 2.0 / (jnp.exp(2.0 * a) + 1.0)
      gate = jnp.where(dots < 0.0, -t, t)
      for q in range(4):
        e0 = g * LANES + q * 4
        rvv = relv[pl.ds(e0 * 4, LANES)]
        g4 = gate.at[q * 4 + lane4].get(mode='promise_in_bounds')
        plsc.store_scatter(stage, [e0 + lane4, lmod4], rvv * g4)
      return 0

    lax.fori_loop(0, ch // LANES, group, 0)
    pltpu.sync_copy(rows_v, agg_sp.at[dst_v], add=True)
    pltpu.sync_copy(stage, vtab_sp.at[dst_v], add=True)
    return 0

  lax.fori_loop(0, ew // ch, chunk, 0)
  plsc.subcore_barrier()

  @pl.when(s == 0)
  def _():
    pltpu.sync_copy(agg_sp.at[pl.ds(0, N_NODES)],
                    agg_out.at[pl.ds(c * N_NODES, N_NODES)])
    pltpu.sync_copy(vtab_sp.at[pl.ds(0, N_NODES)],
                    vd_out.at[pl.ds(c * N_NODES, N_NODES)])


def _sc_edge_phase(hw, ea, src_pad, dst_pad, rel, gv, e_pad, has_ea, ch):
  scratch = [
      pltpu.VMEM((ch,), jnp.int32),
      pltpu.VMEM((ch,), jnp.int32),
      pltpu.VMEM((ch, H), jnp.float32),
      pltpu.VMEM((ch, H) if has_ea else (LANES,), jnp.float32),
      pltpu.VMEM((H,), jnp.float32),
      pltpu.VMEM((4 * ch,), jnp.float32),
      pltpu.VMEM((ch, 4), jnp.float32),
      pltpu.VMEM_SHARED((NROWS, H), jnp.float32),
      pltpu.VMEM_SHARED((NROWS, 4), jnp.float32),
      pltpu.SemaphoreType.DMA,
  ]
  out_type = [
      jax.ShapeDtypeStruct((NC * N_NODES, H), jnp.float32),
      jax.ShapeDtypeStruct((NC * N_NODES, 4), jnp.float32),
  ]
  kfn = pl.kernel(
      functools.partial(_sc_edge_body, has_ea, e_pad, ch),
      out_type=out_type,
      mesh=_sc_mesh(),
      compiler_params=_SC_PARAMS,
      scratch_types=scratch,
  )
  if not has_ea:
    ea = jnp.zeros((LANES,), jnp.float32)
  return kfn(hw, ea, src_pad, dst_pad, rel, gv)


def kernel(x, edge_index_intra, edge_index_inter, pos, edge_attr, batch,
           params):
  p = params
  e_i = _epad(edge_index_intra.shape[1], CH)
  e_s = _epad(edge_index_inter.shape[1], CH_S)
  si_i, di_i = _pad_edges(edge_index_intra[0], edge_index_intra[1], e_i)
  ss_s, ds_s = _pad_edges(edge_index_inter[0], edge_index_inter[1], e_s)
  ea_pad = jnp.concatenate(
      [edge_attr,
       jnp.zeros((e_i - edge_attr.shape[0], D_EDGE), jnp.float32)])
  posp = jnp.concatenate([pos, jnp.zeros((NROWS - N_NODES, 3), jnp.float32)])
  px = posp[:, 0].copy()
  py = posp[:, 1].copy()
  pz = posp[:, 2].copy()

  we_all = jnp.stack([p['We%d' % l] for l in range(4)])
  bi_all = jnp.stack([p['bi%d' % l].reshape(1, H) for l in range(4)])
  eab = _tc_edgefeat(ea_pad, we_all, bi_all, e_i)

  h = _tc_embed(x, p['Wn'], p['bn'])

  rel_i, rel_s, deg_p = _sc_prep(si_i, di_i, ss_s, ds_s, px, py, pz, e_i, e_s)
  recip, scale = _tc_degfin(deg_p.reshape(NC, N_NODES, 4))

  vl = jnp.zeros((N_NODES, 4), jnp.float32)
  vp = jnp.zeros((N_NODES, 4), jnp.float32)
  for l in range(4):
    hwi, hws = _tc_proj(h, p['Wi%d' % l], p['Ws%d' % l], p['bs%d' % l])
    aggi_p, vld_p = _sc_edge_phase(
        hwi, eab[l], si_i, di_i, rel_i, p['gl%d' % l], e_i, True, CH)
    aggs_p, vpd_p = _sc_edge_phase(
        hws, None, ss_s, ds_s, rel_s, p['gp%d' % l], e_s, False, CH_S)
    h, vl, vp = _tc_update(h, aggi_p.reshape(NC, N_NODES, H),
                           aggs_p.reshape(NC, N_NODES, H),
                           vld_p.reshape(NC, N_NODES, 4),
                           vpd_p.reshape(NC, N_NODES, 4), vl, vp,
                           recip, scale)

  fw_all = jnp.stack([p['fW%d' % j] for j in range(3)])
  fb_all = jnp.stack([p['fb%d' % j].reshape(1, H) for j in range(3)])
  gam_all = jnp.stack([p['gamma%d' % j].reshape(1, H) for j in range(3)])
  bet_all = jnp.stack([p['beta%d' % j].reshape(1, H) for j in range(3)])
  out = _tc_head(h, batch.reshape(1, N_NODES), fw_all, fb_all, gam_all,
                 bet_all, p['fWout'], p['fbout'])
  return out.reshape(-1)
